# Initial kernel scaffold; baseline (speedup 1.0000x reference)
import jax
import jax.numpy as jnp
from jax.experimental import pallas as pl


def kernel(x, edge_index, batch, embed, conv_W, conv_b, dense0_W, dense0_b, dense_W, dense_b, final_W, final_b):
    raise NotImplementedError("write your pallas kernel here")


# speedup vs baseline: 12.2188x; 12.2188x over previous
import functools

import jax
import jax.numpy as jnp
from jax import lax
from jax.experimental import pallas as pl
from jax.experimental.pallas import tpu as pltpu
from jax.experimental.pallas import tpu_sc as plsc

F32 = jnp.float32
NEG_INF = float("-inf")

_N = 100000
_E = 1600000
_C = 32
_G = 64
_CONV_LAYERS = 5
_DENSE_LAYERS = 3

_NC = 2
_NS = 16
_LANE = 16

_BLK = 2048
_NP = 100352
_GRID = _NP // _BLK
_NPT = _NP // _NS
_ZCH = 128
_NZ = _NPT // _ZCH

_KCH = 56
_R = 784
_OUTER = _R // _KCH
_RT = _NS * _R
_EP = _RT * 128

_EH = _E // 2
_KCH2 = 56
_R2 = 392
_OUTER2 = _R2 // _KCH2
_RT2 = _NS * _R2
_EP2 = _RT2 * 128


def _fill_rows(ref, nrows, value):
    def body(i, _):
        ref[i, :] = jnp.full((_LANE,), value, F32)
        return 0
    lax.fori_loop(0, nrows, body, 0)


def _sc_mesh():
    return plsc.VectorSubcoreMesh(core_axis_name="c", subcore_axis_name="s")


def _sc_deg(dst2):
    @functools.partial(
        pl.kernel,
        out_type=jax.ShapeDtypeStruct((_NC, _NP, _LANE), F32),
        mesh=_sc_mesh(),
        compiler_params=pltpu.CompilerParams(use_tc_tiling_on_sc=False),
        scratch_types=[
            pltpu.VMEM((_KCH2, 128), jnp.int32),
            pltpu.VMEM((128, _LANE), F32),
            pltpu.VMEM((_ZCH, _LANE), F32),
            pltpu.VMEM_SHARED((_NP, _LANE), F32),
        ],
    )
    def k(dst_hbm, out_hbm, didx, ones_v, zbuf, acc_sh):
        c = lax.axis_index("c")
        s = lax.axis_index("s")
        _fill_rows(zbuf, _ZCH, 0.0)
        _fill_rows(ones_v, 128, 1.0)
        base = s * _NPT

        def zero_chunk(m, _):
            pltpu.sync_copy(zbuf, acc_sh.at[pl.ds(base + m * _ZCH, _ZCH)])
            return 0
        lax.fori_loop(0, _NZ, zero_chunk, 0)
        plsc.subcore_barrier()

        rbase = s * _R2
        for o in range(_OUTER2):
            pltpu.sync_copy(dst_hbm.at[c, pl.ds(rbase + o * _KCH2, _KCH2), :],
                            didx)

            def inner(kk, _):
                pltpu.sync_copy(ones_v, acc_sh.at[didx.at[kk]], add=True)
                return 0
            lax.fori_loop(0, _KCH2, inner, 0)

        plsc.subcore_barrier()

        def copy_out(m, _):
            off = base + m * _ZCH
            pltpu.sync_copy(acc_sh.at[pl.ds(off, _ZCH)], zbuf)
            pltpu.sync_copy(zbuf, out_hbm.at[c, pl.ds(off, _ZCH), :])
            return 0
        lax.fori_loop(0, _NZ, copy_out, 0)

    return k(dst2)


def _sc_edge(hwp, src2, dstr):
    @functools.partial(
        pl.kernel,
        out_type=jax.ShapeDtypeStruct((_NC, _NP, _LANE), F32),
        mesh=_sc_mesh(),
        compiler_params=pltpu.CompilerParams(use_tc_tiling_on_sc=False),
        scratch_types=[
            pltpu.VMEM((_KCH, 128), jnp.int32),
            pltpu.VMEM((_KCH, 128), jnp.int32),
            pltpu.VMEM((128, _LANE), F32),
            pltpu.VMEM((_ZCH, _LANE), F32),
            pltpu.VMEM_SHARED((_NP, _LANE), F32),
            pltpu.SemaphoreType.DMA,
        ],
    )
    def k(hwp_hbm, src_hbm, dst_hbm, out_hbm, sidx, didx, rows, zbuf, acc_sh,
          sem):
        c = lax.axis_index("c")
        s = lax.axis_index("s")
        _fill_rows(zbuf, _ZCH, 0.0)
        base = s * _NPT

        def zero_chunk(m, _):
            pltpu.sync_copy(zbuf, acc_sh.at[pl.ds(base + m * _ZCH, _ZCH)])
            return 0
        lax.fori_loop(0, _NZ, zero_chunk, 0)
        plsc.subcore_barrier()

        rbase = s * _R
        for o in range(_OUTER):
            pltpu.sync_copy(src_hbm.at[c, pl.ds(rbase + o * _KCH, _KCH), :],
                            sidx)
            pltpu.sync_copy(dst_hbm.at[pl.ds(rbase + o * _KCH, _KCH), :],
                            didx)

            def inner(kk, _):
                pltpu.async_copy(hwp_hbm.at[sidx.at[kk]], rows, sem).wait()
                pltpu.sync_copy(rows, acc_sh.at[didx.at[kk]], add=True)
                return 0
            lax.fori_loop(0, _KCH, inner, 0)

        plsc.subcore_barrier()

        def copy_out(m, _):
            off = base + m * _ZCH
            pltpu.sync_copy(acc_sh.at[pl.ds(off, _ZCH)], zbuf)
            pltpu.sync_copy(zbuf, out_hbm.at[c, pl.ds(off, _ZCH), :])
            return 0
        lax.fori_loop(0, _NZ, copy_out, 0)

    return k(hwp, src2, dstr)


def _tc_a_body(x_ref, deg_ref, emb_ref, w_ref, dinv_ref, hwp_ref):
    i = pl.program_id(0)
    dp = deg_ref[0][:, 0:1] + deg_ref[1][:, 0:1]
    rowid = i * _BLK + lax.broadcasted_iota(jnp.int32, (_BLK, 1), 0)
    dinv = jnp.where(rowid < _N, lax.rsqrt(dp + 1.0), 0.0)
    onehot = (x_ref[:] == lax.broadcasted_iota(jnp.int32, (_BLK, _C), 1)
              ).astype(F32)
    emb_w = jnp.dot(emb_ref[:], w_ref[:], preferred_element_type=F32)
    hw = jnp.dot(onehot, emb_w, preferred_element_type=F32)
    hwn = dinv * hw
    dinv_ref[:] = dinv
    hwp_ref[0, :, :] = hwn[:, :_LANE]
    hwp_ref[1, :, :] = hwn[:, _LANE:]


def _tc_a(xp, degp, emb_pad, w0):
    return pl.pallas_call(
        _tc_a_body,
        grid=(_GRID,),
        in_specs=[
            pl.BlockSpec((_BLK, 1), lambda i: (i, 0)),
            pl.BlockSpec((_NC, _BLK, _LANE), lambda i: (0, i, 0)),
            pl.BlockSpec((_C, _C), lambda i: (0, 0)),
            pl.BlockSpec((_C, _C), lambda i: (0, 0)),
        ],
        out_specs=[
            pl.BlockSpec((_BLK, 1), lambda i: (i, 0)),
            pl.BlockSpec((_NC, _BLK, _LANE), lambda i: (0, i, 0)),
        ],
        out_shape=[
            jax.ShapeDtypeStruct((_NP, 1), F32),
            jax.ShapeDtypeStruct((_NC, _NP, _LANE), F32),
        ],
    )(xp, degp, emb_pad, w0)


def _tc_b_body(acc_ref, hwp_ref, dinv_ref, b_ref, w_ref, out_ref):
    acc = jnp.concatenate([acc_ref[0], acc_ref[1]], axis=1)
    hwp = jnp.concatenate([hwp_ref[0], hwp_ref[1]], axis=1)
    dinv = dinv_ref[:]
    h = jnp.maximum(dinv * (acc + hwp) + b_ref[:], 0.0)
    hwn = dinv * jnp.dot(h, w_ref[:], preferred_element_type=F32)
    out_ref[0, :, :] = hwn[:, :_LANE]
    out_ref[1, :, :] = hwn[:, _LANE:]


def _tc_b(acc, hwp, dinv_p, b_row, w_next):
    return pl.pallas_call(
        _tc_b_body,
        grid=(_GRID,),
        in_specs=[
            pl.BlockSpec((_NC, _BLK, _LANE), lambda i: (0, i, 0)),
            pl.BlockSpec((_NC, _BLK, _LANE), lambda i: (0, i, 0)),
            pl.BlockSpec((_BLK, 1), lambda i: (i, 0)),
            pl.BlockSpec((1, _C), lambda i: (0, 0)),
            pl.BlockSpec((_C, _C), lambda i: (0, 0)),
        ],
        out_specs=pl.BlockSpec((_NC, _BLK, _LANE), lambda i: (0, i, 0)),
        out_shape=jax.ShapeDtypeStruct((_NC, _NP, _LANE), F32),
    )(acc, hwp, dinv_p, b_row, w_next)


def _tc_seg_body(acc_ref, hwp_ref, dinv_ref, bat_ref, b_ref, d0w_ref,
                 d0b_ref, dw_ref, db_ref, fw_ref, fb_ref, out_ref, smax_ref):
    i = pl.program_id(0)

    @pl.when(i == 0)
    def _():
        smax_ref[:] = jnp.full((_G + 8, _C), NEG_INF, F32)

    acc = jnp.concatenate([acc_ref[0], acc_ref[1]], axis=1)
    hwp = jnp.concatenate([hwp_ref[0], hwp_ref[1]], axis=1)
    dinv = dinv_ref[:]
    h = jnp.maximum(dinv * (acc + hwp) + b_ref[:], 0.0)

    bi = bat_ref[:]
    g_first = bat_ref[0, 0]
    g_last = bat_ref[_BLK - 1, 0]

    def upd(g, _):
        m = jnp.max(jnp.where(bi == g, h, NEG_INF), axis=0, keepdims=True)
        cur = smax_ref[pl.ds(g, 1), :]
        smax_ref[pl.ds(g, 1), :] = jnp.maximum(cur, m)
        return 0
    lax.fori_loop(g_first, g_last + 1, upd, 0)

    @pl.when(i == _GRID - 1)
    def _():
        g = smax_ref[0:_G, :]
        g = jnp.maximum(
            jnp.dot(g, d0w_ref[:], preferred_element_type=F32) + d0b_ref[:],
            0.0)
        for j in range(_DENSE_LAYERS):
            g = jnp.maximum(
                jnp.dot(g, dw_ref[j], preferred_element_type=F32)
                + db_ref[j], 0.0)
        logits = jnp.dot(g, fw_ref[:], preferred_element_type=F32) + fb_ref[:]
        m = jnp.max(logits, axis=1, keepdims=True)
        z = logits - m
        lse = jnp.log(jnp.sum(jnp.exp(z), axis=1, keepdims=True))
        out_ref[:] = (z - lse)[:, 0:2]


def _tc_seg(acc, hwp, dinv_p, batch_p, b_row, d0w, d0b, dw, db, fw, fb):
    return pl.pallas_call(
        _tc_seg_body,
        grid=(_GRID,),
        in_specs=[
            pl.BlockSpec((_NC, _BLK, _LANE), lambda i: (0, i, 0)),
            pl.BlockSpec((_NC, _BLK, _LANE), lambda i: (0, i, 0)),
            pl.BlockSpec((_BLK, 1), lambda i: (i, 0)),
            pl.BlockSpec((_BLK, 1), lambda i: (i, 0)),
            pl.BlockSpec((1, _C), lambda i: (0, 0)),
            pl.BlockSpec((_C, _C), lambda i: (0, 0)),
            pl.BlockSpec((1, _C), lambda i: (0, 0)),
            pl.BlockSpec((_DENSE_LAYERS, _C, _C), lambda i: (0, 0, 0)),
            pl.BlockSpec((_DENSE_LAYERS, 1, _C), lambda i: (0, 0, 0)),
            pl.BlockSpec((_C, 8), lambda i: (0, 0)),
            pl.BlockSpec((1, 8), lambda i: (0, 0)),
        ],
        out_specs=pl.BlockSpec((_G, 2), lambda i: (0, 0)),
        out_shape=jax.ShapeDtypeStruct((_G, 2), F32),
        scratch_shapes=[pltpu.VMEM((_G + 8, _C), F32)],
    )(acc, hwp, dinv_p, batch_p, b_row, d0w, d0b, dw, db, fw, fb)


def kernel(x, edge_index, batch, embed, conv_W, conv_b, dense0_W, dense0_b,
           dense_W, dense_b, final_W, final_b):
    x32 = x.astype(jnp.int32)
    src = edge_index[0].astype(jnp.int32)
    dst = edge_index[1].astype(jnp.int32)
    bat = batch.astype(jnp.int32)

    xp = jnp.pad(x32, (0, _NP - _N)).reshape(_NP, 1)
    batch_p = jnp.pad(bat, (0, _NP - _N),
                      constant_values=_G).reshape(_NP, 1)

    src_pad = jnp.pad(src, (0, _EP - _E), constant_values=_N)
    src2 = jnp.stack([src_pad, src_pad + _NP]).reshape(_NC, _RT, 128)
    dstr = jnp.pad(dst, (0, _EP - _E),
                   constant_values=_N).reshape(_RT, 128)

    dh0 = jnp.pad(dst[:_EH], (0, _EP2 - _EH), constant_values=_N)
    dh1 = jnp.pad(dst[_EH:], (0, _EP2 - (_E - _EH)), constant_values=_N)
    dst2 = jnp.stack([dh0, dh1]).reshape(_NC, _RT2, 128)

    emb_pad = jnp.zeros((_C, _C), F32).at[:embed.shape[0]].set(embed)
    b_rows = conv_b.reshape(_CONV_LAYERS, 1, _C)
    d0b = dense0_b.reshape(1, _C)
    db = dense_b.reshape(_DENSE_LAYERS, 1, _C)
    fw = jnp.zeros((_C, 8), F32).at[:, :2].set(final_W)
    fb = jnp.full((1, 8), -1e30, F32).at[0, :2].set(final_b)

    degp = _sc_deg(dst2)
    dinv_p, hwp = _tc_a(xp, degp, emb_pad, conv_W[0])

    for l in range(_CONV_LAYERS):
        acc = _sc_edge(hwp.reshape(_NC * _NP, _LANE), src2, dstr)
        if l + 1 < _CONV_LAYERS:
            hwp = _tc_b(acc, hwp, dinv_p, b_rows[l], conv_W[l + 1])
        else:
            out = _tc_seg(acc, hwp, dinv_p, batch_p, b_rows[l], dense0_W,
                          d0b, dense_W, db, fw, fb)
    return out


# speedup vs baseline: 22.1609x; 1.8137x over previous
import functools

import jax
import jax.numpy as jnp
from jax import lax
from jax.experimental import pallas as pl
from jax.experimental.pallas import tpu as pltpu
from jax.experimental.pallas import tpu_sc as plsc

F32 = jnp.float32
NEG_INF = float("-inf")

_N = 100000
_E = 1600000
_C = 32
_G = 64
_CONV_LAYERS = 5
_DENSE_LAYERS = 3

_NC = 2
_NS = 16
_LANE = 16

_BLK = 2048
_NP = 100352
_GRID = _NP // _BLK
_NPT = _NP // _NS
_ZCH = 128
_NZ = _NPT // _ZCH

_KCH = 56
_R = 784
_OUTER = _R // _KCH
_RT = _NS * _R
_EP = _RT * 128

_EH = _E // 2
_KCH2 = 56
_R2 = 392
_OUTER2 = _R2 // _KCH2
_RT2 = _NS * _R2
_EP2 = _RT2 * 128


def _fill_rows(ref, nrows, value):
    def body(i, _):
        ref[i, :] = jnp.full((_LANE,), value, F32)
        return 0
    lax.fori_loop(0, nrows, body, 0)


def _sc_mesh():
    return plsc.VectorSubcoreMesh(core_axis_name="c", subcore_axis_name="s")


def _sc_deg(dst2):
    @functools.partial(
        pl.kernel,
        out_type=jax.ShapeDtypeStruct((_NC, _NP, _LANE), F32),
        mesh=_sc_mesh(),
        compiler_params=pltpu.CompilerParams(use_tc_tiling_on_sc=False),
        scratch_types=[
            pltpu.VMEM((_KCH2, 128), jnp.int32),
            pltpu.VMEM((128, _LANE), F32),
            pltpu.VMEM((_ZCH, _LANE), F32),
            pltpu.VMEM_SHARED((_NP, _LANE), F32),
        ],
    )
    def k(dst_hbm, out_hbm, didx, ones_v, zbuf, acc_sh):
        c = lax.axis_index("c")
        s = lax.axis_index("s")
        _fill_rows(zbuf, _ZCH, 0.0)
        _fill_rows(ones_v, 128, 1.0)
        base = s * _NPT

        def zero_chunk(m, _):
            pltpu.sync_copy(zbuf, acc_sh.at[pl.ds(base + m * _ZCH, _ZCH)])
            return 0
        lax.fori_loop(0, _NZ, zero_chunk, 0)
        plsc.subcore_barrier()

        rbase = s * _R2
        for o in range(_OUTER2):
            pltpu.sync_copy(dst_hbm.at[c, pl.ds(rbase + o * _KCH2, _KCH2), :],
                            didx)

            def inner(kk, _):
                pltpu.sync_copy(ones_v, acc_sh.at[didx.at[kk]], add=True)
                return 0
            lax.fori_loop(0, _KCH2, inner, 0)

        plsc.subcore_barrier()

        def copy_out(m, _):
            off = base + m * _ZCH
            pltpu.sync_copy(acc_sh.at[pl.ds(off, _ZCH)], zbuf)
            pltpu.sync_copy(zbuf, out_hbm.at[c, pl.ds(off, _ZCH), :])
            return 0
        lax.fori_loop(0, _NZ, copy_out, 0)

    return k(dst2)


_NBUF = 4
_NGRP = _KCH // _NBUF


def _sc_edge(hwp, src2, dstr):
    @functools.partial(
        pl.kernel,
        out_type=jax.ShapeDtypeStruct((_NC, _NP, _LANE), F32),
        mesh=_sc_mesh(),
        compiler_params=pltpu.CompilerParams(use_tc_tiling_on_sc=False),
        scratch_types=[
            pltpu.VMEM((_KCH, 128), jnp.int32),
            pltpu.VMEM((_KCH, 128), jnp.int32),
            pltpu.VMEM((_NBUF, 128, _LANE), F32),
            pltpu.VMEM((_ZCH, _LANE), F32),
            pltpu.VMEM_SHARED((_NP, _LANE), F32),
            pltpu.SemaphoreType.DMA((_NBUF,)),
            pltpu.SemaphoreType.DMA((_NBUF,)),
        ],
    )
    def k(hwp_hbm, src_hbm, dst_hbm, out_hbm, sidx, didx, rows, zbuf, acc_sh,
          gsem, ssem):
        c = lax.axis_index("c")
        s = lax.axis_index("s")
        _fill_rows(zbuf, _ZCH, 0.0)
        base = s * _NPT

        def zero_chunk(m, _):
            pltpu.sync_copy(zbuf, acc_sh.at[pl.ds(base + m * _ZCH, _ZCH)])
            return 0
        lax.fori_loop(0, _NZ, zero_chunk, 0)
        plsc.subcore_barrier()

        rbase = s * _R
        for o in range(_OUTER):
            pltpu.sync_copy(src_hbm.at[c, pl.ds(rbase + o * _KCH, _KCH), :],
                            sidx)
            pltpu.sync_copy(dst_hbm.at[pl.ds(rbase + o * _KCH, _KCH), :],
                            didx)

            for b in range(_NBUF):
                pltpu.async_copy(hwp_hbm.at[sidx.at[b]], rows.at[b],
                                 gsem.at[b])

            def group(g, _):
                for b in range(_NBUF):
                    j = g * _NBUF + b
                    pltpu.make_async_copy(hwp_hbm.at[sidx.at[j]], rows.at[b],
                                          gsem.at[b]).wait()
                    pltpu.async_copy(rows.at[b], acc_sh.at[didx.at[j]],
                                     ssem.at[b], add=True)
                    pltpu.make_async_copy(rows.at[b],
                                          acc_sh.at[didx.at[j]],
                                          ssem.at[b]).wait()
                    pltpu.async_copy(hwp_hbm.at[sidx.at[j + _NBUF]],
                                     rows.at[b], gsem.at[b])
                return 0
            lax.fori_loop(0, _NGRP - 1, group, 0)

            for b in range(_NBUF):
                j = (_NGRP - 1) * _NBUF + b
                pltpu.make_async_copy(hwp_hbm.at[sidx.at[j]], rows.at[b],
                                      gsem.at[b]).wait()
                pltpu.sync_copy(rows.at[b], acc_sh.at[didx.at[j]], add=True)

        plsc.subcore_barrier()

        def copy_out(m, _):
            off = base + m * _ZCH
            pltpu.sync_copy(acc_sh.at[pl.ds(off, _ZCH)], zbuf)
            pltpu.sync_copy(zbuf, out_hbm.at[c, pl.ds(off, _ZCH), :])
            return 0
        lax.fori_loop(0, _NZ, copy_out, 0)

    return k(hwp, src2, dstr)


def _tc_a_body(x_ref, deg_ref, emb_ref, w_ref, dinv_ref, hwp_ref):
    i = pl.program_id(0)
    dp = deg_ref[0][:, 0:1] + deg_ref[1][:, 0:1]
    rowid = i * _BLK + lax.broadcasted_iota(jnp.int32, (_BLK, 1), 0)
    dinv = jnp.where(rowid < _N, lax.rsqrt(dp + 1.0), 0.0)
    onehot = (x_ref[:] == lax.broadcasted_iota(jnp.int32, (_BLK, _C), 1)
              ).astype(F32)
    emb_w = jnp.dot(emb_ref[:], w_ref[:], preferred_element_type=F32)
    hw = jnp.dot(onehot, emb_w, preferred_element_type=F32)
    hwn = dinv * hw
    dinv_ref[:] = dinv
    hwp_ref[0, :, :] = hwn[:, :_LANE]
    hwp_ref[1, :, :] = hwn[:, _LANE:]


def _tc_a(xp, degp, emb_pad, w0):
    return pl.pallas_call(
        _tc_a_body,
        grid=(_GRID,),
        in_specs=[
            pl.BlockSpec((_BLK, 1), lambda i: (i, 0)),
            pl.BlockSpec((_NC, _BLK, _LANE), lambda i: (0, i, 0)),
            pl.BlockSpec((_C, _C), lambda i: (0, 0)),
            pl.BlockSpec((_C, _C), lambda i: (0, 0)),
        ],
        out_specs=[
            pl.BlockSpec((_BLK, 1), lambda i: (i, 0)),
            pl.BlockSpec((_NC, _BLK, _LANE), lambda i: (0, i, 0)),
        ],
        out_shape=[
            jax.ShapeDtypeStruct((_NP, 1), F32),
            jax.ShapeDtypeStruct((_NC, _NP, _LANE), F32),
        ],
    )(xp, degp, emb_pad, w0)


def _tc_b_body(acc_ref, hwp_ref, dinv_ref, b_ref, w_ref, out_ref):
    acc = jnp.concatenate([acc_ref[0], acc_ref[1]], axis=1)
    hwp = jnp.concatenate([hwp_ref[0], hwp_ref[1]], axis=1)
    dinv = dinv_ref[:]
    h = jnp.maximum(dinv * (acc + hwp) + b_ref[:], 0.0)
    hwn = dinv * jnp.dot(h, w_ref[:], preferred_element_type=F32)
    out_ref[0, :, :] = hwn[:, :_LANE]
    out_ref[1, :, :] = hwn[:, _LANE:]


def _tc_b(acc, hwp, dinv_p, b_row, w_next):
    return pl.pallas_call(
        _tc_b_body,
        grid=(_GRID,),
        in_specs=[
            pl.BlockSpec((_NC, _BLK, _LANE), lambda i: (0, i, 0)),
            pl.BlockSpec((_NC, _BLK, _LANE), lambda i: (0, i, 0)),
            pl.BlockSpec((_BLK, 1), lambda i: (i, 0)),
            pl.BlockSpec((1, _C), lambda i: (0, 0)),
            pl.BlockSpec((_C, _C), lambda i: (0, 0)),
        ],
        out_specs=pl.BlockSpec((_NC, _BLK, _LANE), lambda i: (0, i, 0)),
        out_shape=jax.ShapeDtypeStruct((_NC, _NP, _LANE), F32),
    )(acc, hwp, dinv_p, b_row, w_next)


def _tc_seg_body(acc_ref, hwp_ref, dinv_ref, bat_ref, b_ref, d0w_ref,
                 d0b_ref, dw_ref, db_ref, fw_ref, fb_ref, out_ref, smax_ref):
    i = pl.program_id(0)

    @pl.when(i == 0)
    def _():
        smax_ref[:] = jnp.full((_G + 8, _C), NEG_INF, F32)

    acc = jnp.concatenate([acc_ref[0], acc_ref[1]], axis=1)
    hwp = jnp.concatenate([hwp_ref[0], hwp_ref[1]], axis=1)
    dinv = dinv_ref[:]
    h = jnp.maximum(dinv * (acc + hwp) + b_ref[:], 0.0)

    bi = bat_ref[:]
    g_first = bat_ref[0, 0]
    g_last = bat_ref[_BLK - 1, 0]

    def upd(g, _):
        m = jnp.max(jnp.where(bi == g, h, NEG_INF), axis=0, keepdims=True)
        cur = smax_ref[pl.ds(g, 1), :]
        smax_ref[pl.ds(g, 1), :] = jnp.maximum(cur, m)
        return 0
    lax.fori_loop(g_first, g_last + 1, upd, 0)

    @pl.when(i == _GRID - 1)
    def _():
        g = smax_ref[0:_G, :]
        g = jnp.maximum(
            jnp.dot(g, d0w_ref[:], preferred_element_type=F32) + d0b_ref[:],
            0.0)
        for j in range(_DENSE_LAYERS):
            g = jnp.maximum(
                jnp.dot(g, dw_ref[j], preferred_element_type=F32)
                + db_ref[j], 0.0)
        logits = jnp.dot(g, fw_ref[:], preferred_element_type=F32) + fb_ref[:]
        m = jnp.max(logits, axis=1, keepdims=True)
        z = logits - m
        lse = jnp.log(jnp.sum(jnp.exp(z), axis=1, keepdims=True))
        out_ref[:] = (z - lse)[:, 0:2]


def _tc_seg(acc, hwp, dinv_p, batch_p, b_row, d0w, d0b, dw, db, fw, fb):
    return pl.pallas_call(
        _tc_seg_body,
        grid=(_GRID,),
        in_specs=[
            pl.BlockSpec((_NC, _BLK, _LANE), lambda i: (0, i, 0)),
            pl.BlockSpec((_NC, _BLK, _LANE), lambda i: (0, i, 0)),
            pl.BlockSpec((_BLK, 1), lambda i: (i, 0)),
            pl.BlockSpec((_BLK, 1), lambda i: (i, 0)),
            pl.BlockSpec((1, _C), lambda i: (0, 0)),
            pl.BlockSpec((_C, _C), lambda i: (0, 0)),
            pl.BlockSpec((1, _C), lambda i: (0, 0)),
            pl.BlockSpec((_DENSE_LAYERS, _C, _C), lambda i: (0, 0, 0)),
            pl.BlockSpec((_DENSE_LAYERS, 1, _C), lambda i: (0, 0, 0)),
            pl.BlockSpec((_C, 8), lambda i: (0, 0)),
            pl.BlockSpec((1, 8), lambda i: (0, 0)),
        ],
        out_specs=pl.BlockSpec((_G, 2), lambda i: (0, 0)),
        out_shape=jax.ShapeDtypeStruct((_G, 2), F32),
        scratch_shapes=[pltpu.VMEM((_G + 8, _C), F32)],
    )(acc, hwp, dinv_p, batch_p, b_row, d0w, d0b, dw, db, fw, fb)


def kernel(x, edge_index, batch, embed, conv_W, conv_b, dense0_W, dense0_b,
           dense_W, dense_b, final_W, final_b):
    x32 = x.astype(jnp.int32)
    src = edge_index[0].astype(jnp.int32)
    dst = edge_index[1].astype(jnp.int32)
    bat = batch.astype(jnp.int32)

    xp = jnp.pad(x32, (0, _NP - _N)).reshape(_NP, 1)
    batch_p = jnp.pad(bat, (0, _NP - _N),
                      constant_values=_G).reshape(_NP, 1)

    src_pad = jnp.pad(src, (0, _EP - _E), constant_values=_N)
    src2 = jnp.stack([src_pad, src_pad + _NP]).reshape(_NC, _RT, 128)
    dstr = jnp.pad(dst, (0, _EP - _E),
                   constant_values=_N).reshape(_RT, 128)

    dh0 = jnp.pad(dst[:_EH], (0, _EP2 - _EH), constant_values=_N)
    dh1 = jnp.pad(dst[_EH:], (0, _EP2 - (_E - _EH)), constant_values=_N)
    dst2 = jnp.stack([dh0, dh1]).reshape(_NC, _RT2, 128)

    emb_pad = jnp.zeros((_C, _C), F32).at[:embed.shape[0]].set(embed)
    b_rows = conv_b.reshape(_CONV_LAYERS, 1, _C)
    d0b = dense0_b.reshape(1, _C)
    db = dense_b.reshape(_DENSE_LAYERS, 1, _C)
    fw = jnp.zeros((_C, 8), F32).at[:, :2].set(final_W)
    fb = jnp.full((1, 8), -1e30, F32).at[0, :2].set(final_b)

    degp = _sc_deg(dst2)
    dinv_p, hwp = _tc_a(xp, degp, emb_pad, conv_W[0])

    for l in range(_CONV_LAYERS):
        acc = _sc_edge(hwp.reshape(_NC * _NP, _LANE), src2, dstr)
        if l + 1 < _CONV_LAYERS:
            hwp = _tc_b(acc, hwp, dinv_p, b_rows[l], conv_W[l + 1])
        else:
            out = _tc_seg(acc, hwp, dinv_p, batch_p, b_rows[l], dense0_W,
                          d0b, dense_W, db, fw, fb)
    return out


# speedup vs baseline: 35.9970x; 1.6243x over previous
import functools

import jax
import jax.numpy as jnp
from jax import lax
from jax.experimental import pallas as pl
from jax.experimental.pallas import tpu as pltpu
from jax.experimental.pallas import tpu_sc as plsc

F32 = jnp.float32
NEG_INF = float("-inf")

_N = 100000
_E = 1600000
_C = 32
_G = 64
_CONV_LAYERS = 5
_DENSE_LAYERS = 3

_NC = 2
_NS = 16
_LANE = 16

_BLK = 2048
_NP = 100352
_GRID = _NP // _BLK
_NPT = _NP // _NS
_ZCH = 128
_NZ = _NPT // _ZCH

_KCH = 56
_R = 784
_OUTER = _R // _KCH
_RT = _NS * _R
_EP = _RT * 128

_EH = _E // 2
_KCH2 = 56
_R2 = 392
_OUTER2 = _R2 // _KCH2
_RT2 = _NS * _R2
_EP2 = _RT2 * 128


def _fill_rows(ref, nrows, value):
    def body(i, _):
        ref[i, :] = jnp.full((_LANE,), value, F32)
        return 0
    lax.fori_loop(0, nrows, body, 0)


def _sc_mesh():
    return plsc.VectorSubcoreMesh(core_axis_name="c", subcore_axis_name="s")


def _sc_deg(dst2):
    @functools.partial(
        pl.kernel,
        out_type=jax.ShapeDtypeStruct((_NC, _NP, _LANE), F32),
        mesh=_sc_mesh(),
        compiler_params=pltpu.CompilerParams(use_tc_tiling_on_sc=False),
        scratch_types=[
            pltpu.VMEM((_KCH2, 128), jnp.int32),
            pltpu.VMEM((128, _LANE), F32),
            pltpu.VMEM((_ZCH, _LANE), F32),
            pltpu.VMEM_SHARED((_NP, _LANE), F32),
        ],
    )
    def k(dst_hbm, out_hbm, didx, ones_v, zbuf, acc_sh):
        c = lax.axis_index("c")
        s = lax.axis_index("s")
        _fill_rows(zbuf, _ZCH, 0.0)
        _fill_rows(ones_v, 128, 1.0)
        base = s * _NPT

        def zero_chunk(m, _):
            pltpu.sync_copy(zbuf, acc_sh.at[pl.ds(base + m * _ZCH, _ZCH)])
            return 0
        lax.fori_loop(0, _NZ, zero_chunk, 0)
        plsc.subcore_barrier()

        rbase = s * _R2
        for o in range(_OUTER2):
            pltpu.sync_copy(dst_hbm.at[c, pl.ds(rbase + o * _KCH2, _KCH2), :],
                            didx)

            def inner(kk, _):
                pltpu.sync_copy(ones_v, acc_sh.at[didx.at[kk]], add=True)
                return 0
            lax.fori_loop(0, _KCH2, inner, 0)

        plsc.subcore_barrier()

        def copy_out(m, _):
            off = base + m * _ZCH
            pltpu.sync_copy(acc_sh.at[pl.ds(off, _ZCH)], zbuf)
            pltpu.sync_copy(zbuf, out_hbm.at[c, pl.ds(off, _ZCH), :])
            return 0
        lax.fori_loop(0, _NZ, copy_out, 0)

    return k(dst2)


_NBUF = 4
_NGRP = _KCH // _NBUF


def _sc_edge(hwp, src2, dstr):
    @functools.partial(
        pl.kernel,
        out_type=jax.ShapeDtypeStruct((_NC, _NP, _LANE), F32),
        mesh=_sc_mesh(),
        compiler_params=pltpu.CompilerParams(use_tc_tiling_on_sc=False),
        scratch_types=[
            pltpu.VMEM((_KCH, 128), jnp.int32),
            pltpu.VMEM((_KCH, 128), jnp.int32),
            pltpu.VMEM((_NBUF, 128, _LANE), F32),
            pltpu.VMEM((_ZCH, _LANE), F32),
            pltpu.VMEM_SHARED((_NP, _LANE), F32),
            pltpu.SemaphoreType.DMA((_NBUF,)),
            pltpu.SemaphoreType.DMA((_NBUF,)),
        ],
    )
    def k(hwp_hbm, src_hbm, dst_hbm, out_hbm, sidx, didx, rows, zbuf, acc_sh,
          gsem, ssem):
        c = lax.axis_index("c")
        s = lax.axis_index("s")
        _fill_rows(zbuf, _ZCH, 0.0)
        base = s * _NPT

        def zero_chunk(m, _):
            pltpu.sync_copy(zbuf, acc_sh.at[pl.ds(base + m * _ZCH, _ZCH)])
            return 0
        lax.fori_loop(0, _NZ, zero_chunk, 0)
        plsc.subcore_barrier()

        rbase = s * _R
        for o in range(_OUTER):
            pltpu.sync_copy(src_hbm.at[c, pl.ds(rbase + o * _KCH, _KCH), :],
                            sidx)
            pltpu.sync_copy(dst_hbm.at[pl.ds(rbase + o * _KCH, _KCH), :],
                            didx)

            for b in range(_NBUF):
                pltpu.async_copy(hwp_hbm.at[sidx.at[b]], rows.at[b],
                                 gsem.at[b])

            def group(g, _):
                for b in range(_NBUF):
                    j = g * _NBUF + b
                    pltpu.make_async_copy(hwp_hbm.at[sidx.at[j]], rows.at[b],
                                          gsem.at[b]).wait()
                    pltpu.async_copy(rows.at[b], acc_sh.at[didx.at[j]],
                                     ssem.at[b], add=True)
                    pltpu.make_async_copy(rows.at[b],
                                          acc_sh.at[didx.at[j]],
                                          ssem.at[b]).wait()
                    pltpu.async_copy(hwp_hbm.at[sidx.at[j + _NBUF]],
                                     rows.at[b], gsem.at[b])
                return 0
            lax.fori_loop(0, _NGRP - 1, group, 0)

            for b in range(_NBUF):
                j = (_NGRP - 1) * _NBUF + b
                pltpu.make_async_copy(hwp_hbm.at[sidx.at[j]], rows.at[b],
                                      gsem.at[b]).wait()
                pltpu.sync_copy(rows.at[b], acc_sh.at[didx.at[j]], add=True)

        plsc.subcore_barrier()

        def copy_out(m, _):
            off = base + m * _ZCH
            pltpu.sync_copy(acc_sh.at[pl.ds(off, _ZCH)], zbuf)
            pltpu.sync_copy(zbuf, out_hbm.at[c, pl.ds(off, _ZCH), :])
            return 0
        lax.fori_loop(0, _NZ, copy_out, 0)

    return k(hwp, src2, dstr)


_NPB = _NP // 8
_BLKP = _BLK // 8
_FLAV = 17


def _tc_a_body(x_ref, deg_ref, emb_ref, w_ref, dinv_ref, hwp_ref):
    i = pl.program_id(0)
    dp = deg_ref[0] + deg_ref[1]
    row_iota = lax.broadcasted_iota(jnp.int32, (_BLKP, 128), 0)
    lane_iota = lax.broadcasted_iota(jnp.int32, (_BLKP, 128), 1)
    nid = 8 * (i * _BLKP + row_iota) + lane_iota // _LANE
    dinv = jnp.where(nid < _N, lax.rsqrt(dp + 1.0), 0.0)
    ew = jnp.dot(emb_ref[:], w_ref[:], preferred_element_type=F32)
    ew0 = jnp.concatenate([ew[:, :_LANE]] * 8, axis=1)
    ew1 = jnp.concatenate([ew[:, _LANE:]] * 8, axis=1)
    xq = x_ref[:]
    h0 = jnp.zeros((_BLKP, 128), F32)
    h1 = jnp.zeros((_BLKP, 128), F32)
    for f in range(_FLAV):
        sel = xq == f
        h0 = jnp.where(sel, ew0[f:f + 1, :], h0)
        h1 = jnp.where(sel, ew1[f:f + 1, :], h1)
    dinv_ref[:] = dinv
    hwp_ref[0, :, :] = dinv * h0
    hwp_ref[1, :, :] = dinv * h1


def _tc_a(xpk, degp, emb_pad, w0):
    return pl.pallas_call(
        _tc_a_body,
        grid=(_GRID,),
        in_specs=[
            pl.BlockSpec((_BLKP, 128), lambda i: (i, 0)),
            pl.BlockSpec((_NC, _BLKP, 128), lambda i: (0, i, 0)),
            pl.BlockSpec((_C, _C), lambda i: (0, 0)),
            pl.BlockSpec((_C, _C), lambda i: (0, 0)),
        ],
        out_specs=[
            pl.BlockSpec((_BLKP, 128), lambda i: (i, 0)),
            pl.BlockSpec((_NC, _BLKP, 128), lambda i: (0, i, 0)),
        ],
        out_shape=[
            jax.ShapeDtypeStruct((_NPB, 128), F32),
            jax.ShapeDtypeStruct((_NC, _NPB, 128), F32),
        ],
    )(xpk, degp, emb_pad, w0)


def _layer_h(acc_ref, hwp_ref, dinv_ref, b_ref):
    dinv = dinv_ref[:]
    h0 = jnp.maximum(dinv * (acc_ref[0] + hwp_ref[0]) + b_ref[0:1, :], 0.0)
    h1 = jnp.maximum(dinv * (acc_ref[1] + hwp_ref[1]) + b_ref[1:2, :], 0.0)
    return dinv, h0, h1


def _tc_b_body(acc_ref, hwp_ref, dinv_ref, b_ref, wk_ref, out_ref):
    dinv, h0, h1 = _layer_h(acc_ref, hwp_ref, dinv_ref, b_ref)
    hw0 = (jnp.dot(h0, wk_ref[0], preferred_element_type=F32)
           + jnp.dot(h1, wk_ref[2], preferred_element_type=F32))
    hw1 = (jnp.dot(h0, wk_ref[1], preferred_element_type=F32)
           + jnp.dot(h1, wk_ref[3], preferred_element_type=F32))
    out_ref[0, :, :] = dinv * hw0
    out_ref[1, :, :] = dinv * hw1


def _tc_b(acc, hwp, dinvp, bt, wk):
    return pl.pallas_call(
        _tc_b_body,
        grid=(_GRID,),
        in_specs=[
            pl.BlockSpec((_NC, _BLKP, 128), lambda i: (0, i, 0)),
            pl.BlockSpec((_NC, _BLKP, 128), lambda i: (0, i, 0)),
            pl.BlockSpec((_BLKP, 128), lambda i: (i, 0)),
            pl.BlockSpec((2, 128), lambda i: (0, 0)),
            pl.BlockSpec((4, 128, 128), lambda i: (0, 0, 0)),
        ],
        out_specs=pl.BlockSpec((_NC, _BLKP, 128), lambda i: (0, i, 0)),
        out_shape=jax.ShapeDtypeStruct((_NC, _NPB, 128), F32),
    )(acc, hwp, dinvp, bt, wk)


def _tc_seg_body(acc_ref, hwp_ref, dinv_ref, bat_ref, b_ref, d0w_ref,
                 d0b_ref, dw_ref, db_ref, fw_ref, fb_ref, out_ref, smax_ref):
    i = pl.program_id(0)

    @pl.when(i == 0)
    def _():
        smax_ref[:] = jnp.full((_G + 8, _C), NEG_INF, F32)

    _, h0, h1 = _layer_h(acc_ref, hwp_ref, dinv_ref, b_ref)

    bi = bat_ref[:]
    g_first = bat_ref[0, 0]
    g_last = bat_ref[_BLKP - 1, 127]

    def upd(g, _):
        m0 = jnp.max(jnp.where(bi == g, h0, NEG_INF), axis=0, keepdims=True)
        m1 = jnp.max(jnp.where(bi == g, h1, NEG_INF), axis=0, keepdims=True)
        r0 = m0[:, 0:_LANE]
        r1 = m1[:, 0:_LANE]
        for k in range(1, 8):
            r0 = jnp.maximum(r0, m0[:, k * _LANE:(k + 1) * _LANE])
            r1 = jnp.maximum(r1, m1[:, k * _LANE:(k + 1) * _LANE])
        m = jnp.concatenate([r0, r1], axis=1)
        cur = smax_ref[pl.ds(g, 1), :]
        smax_ref[pl.ds(g, 1), :] = jnp.maximum(cur, m)
        return 0
    lax.fori_loop(g_first, g_last + 1, upd, 0)

    @pl.when(i == _GRID - 1)
    def _():
        g = smax_ref[0:_G, :]
        g = jnp.maximum(
            jnp.dot(g, d0w_ref[:], preferred_element_type=F32) + d0b_ref[:],
            0.0)
        for j in range(_DENSE_LAYERS):
            g = jnp.maximum(
                jnp.dot(g, dw_ref[j], preferred_element_type=F32)
                + db_ref[j], 0.0)
        logits = jnp.dot(g, fw_ref[:], preferred_element_type=F32) + fb_ref[:]
        m = jnp.max(logits, axis=1, keepdims=True)
        z = logits - m
        lse = jnp.log(jnp.sum(jnp.exp(z), axis=1, keepdims=True))
        out_ref[:] = (z - lse)[:, 0:2]


def _tc_seg(acc, hwp, dinvp, batpk, bt, d0w, d0b, dw, db, fw, fb):
    return pl.pallas_call(
        _tc_seg_body,
        grid=(_GRID,),
        in_specs=[
            pl.BlockSpec((_NC, _BLKP, 128), lambda i: (0, i, 0)),
            pl.BlockSpec((_NC, _BLKP, 128), lambda i: (0, i, 0)),
            pl.BlockSpec((_BLKP, 128), lambda i: (i, 0)),
            pl.BlockSpec((_BLKP, 128), lambda i: (i, 0)),
            pl.BlockSpec((2, 128), lambda i: (0, 0)),
            pl.BlockSpec((_C, _C), lambda i: (0, 0)),
            pl.BlockSpec((1, _C), lambda i: (0, 0)),
            pl.BlockSpec((_DENSE_LAYERS, _C, _C), lambda i: (0, 0, 0)),
            pl.BlockSpec((_DENSE_LAYERS, 1, _C), lambda i: (0, 0, 0)),
            pl.BlockSpec((_C, 8), lambda i: (0, 0)),
            pl.BlockSpec((1, 8), lambda i: (0, 0)),
        ],
        out_specs=pl.BlockSpec((_G, 2), lambda i: (0, 0)),
        out_shape=jax.ShapeDtypeStruct((_G, 2), F32),
        scratch_shapes=[pltpu.VMEM((_G + 8, _C), F32)],
    )(acc, hwp, dinvp, batpk, bt, d0w, d0b, dw, db, fw, fb)


def _pack_scalar(v, pad_value):
    vp = jnp.pad(v, (0, _NP - _N), constant_values=pad_value)
    return jnp.repeat(vp, _LANE).reshape(_NPB, 128)


def kernel(x, edge_index, batch, embed, conv_W, conv_b, dense0_W, dense0_b,
           dense_W, dense_b, final_W, final_b):
    x32 = x.astype(jnp.int32)
    src = edge_index[0].astype(jnp.int32)
    dst = edge_index[1].astype(jnp.int32)
    bat = batch.astype(jnp.int32)

    xpk = _pack_scalar(x32, 0)
    batpk = _pack_scalar(bat, _G)

    src_pad = jnp.pad(src, (0, _EP - _E), constant_values=_N)
    src2 = jnp.stack([src_pad, src_pad + _NP]).reshape(_NC, _RT, 128)
    dstr = jnp.pad(dst, (0, _EP - _E),
                   constant_values=_N).reshape(_RT, 128)

    dh0 = jnp.pad(dst[:_EH], (0, _EP2 - _EH), constant_values=_N)
    dh1 = jnp.pad(dst[_EH:], (0, _EP2 - (_E - _EH)), constant_values=_N)
    dst2 = jnp.stack([dh0, dh1]).reshape(_NC, _RT2, 128)

    emb_pad = jnp.zeros((_C, _C), F32).at[:embed.shape[0]].set(embed)

    eye8 = jnp.eye(8, dtype=F32)
    wks = []
    for l in range(1, _CONV_LAYERS):
        w = conv_W[l]
        wks.append(jnp.stack([
            jnp.kron(eye8, w[:_LANE, :_LANE]),
            jnp.kron(eye8, w[:_LANE, _LANE:]),
            jnp.kron(eye8, w[_LANE:, :_LANE]),
            jnp.kron(eye8, w[_LANE:, _LANE:]),
        ]))
    bts = [jnp.stack([jnp.tile(conv_b[l][:_LANE], 8),
                      jnp.tile(conv_b[l][_LANE:], 8)])
           for l in range(_CONV_LAYERS)]

    d0b = dense0_b.reshape(1, _C)
    db = dense_b.reshape(_DENSE_LAYERS, 1, _C)
    fw = jnp.zeros((_C, 8), F32).at[:, :2].set(final_W)
    fb = jnp.full((1, 8), -1e30, F32).at[0, :2].set(final_b)

    degp = _sc_deg(dst2)
    dinvp, hwp = _tc_a(xpk, degp.reshape(_NC, _NPB, 128), emb_pad, conv_W[0])

    for l in range(_CONV_LAYERS):
        acc = _sc_edge(hwp.reshape(_NC * _NP, _LANE), src2, dstr)
        accp = acc.reshape(_NC, _NPB, 128)
        if l + 1 < _CONV_LAYERS:
            hwp = _tc_b(accp, hwp, dinvp, bts[l], wks[l])
        else:
            out = _tc_seg(accp, hwp, dinvp, batpk, bts[l], dense0_W,
                          d0b, dense_W, db, fw, fb)
    return out


# speedup vs baseline: 37.8927x; 1.0527x over previous
import functools

import jax
import jax.numpy as jnp
from jax import lax
from jax.experimental import pallas as pl
from jax.experimental.pallas import tpu as pltpu
from jax.experimental.pallas import tpu_sc as plsc

F32 = jnp.float32
NEG_INF = float("-inf")

_N = 100000
_E = 1600000
_C = 32
_G = 64
_CONV_LAYERS = 5
_DENSE_LAYERS = 3

_NC = 2
_NS = 16
_LANE = 16

_BLK = 2048
_NP = 100352
_GRID = _NP // _BLK
_NPT = _NP // _NS
_ZCH = 128
_NZ = _NPT // _ZCH

_KCH = 56
_R = 784
_OUTER = _R // _KCH
_RT = _NS * _R
_EP = _RT * 128

_EH = _E // 2
_KCH2 = 56
_R2 = 392
_OUTER2 = _R2 // _KCH2
_RT2 = _NS * _R2
_EP2 = _RT2 * 128


def _fill_rows(ref, nrows, value):
    def body(i, _):
        ref[i, :] = jnp.full((_LANE,), value, F32)
        return 0
    lax.fori_loop(0, nrows, body, 0)


def _sc_mesh():
    return plsc.VectorSubcoreMesh(core_axis_name="c", subcore_axis_name="s")


def _sc_deg(dst2):
    @functools.partial(
        pl.kernel,
        out_type=jax.ShapeDtypeStruct((_NC, _NP, _LANE), F32),
        mesh=_sc_mesh(),
        compiler_params=pltpu.CompilerParams(use_tc_tiling_on_sc=False),
        scratch_types=[
            pltpu.VMEM((_KCH2, 128), jnp.int32),
            pltpu.VMEM((128, _LANE), F32),
            pltpu.VMEM((_ZCH, _LANE), F32),
            pltpu.VMEM_SHARED((_NP, _LANE), F32),
        ],
    )
    def k(dst_hbm, out_hbm, didx, ones_v, zbuf, acc_sh):
        c = lax.axis_index("c")
        s = lax.axis_index("s")
        _fill_rows(zbuf, _ZCH, 0.0)
        _fill_rows(ones_v, 128, 1.0)
        base = s * _NPT

        def zero_chunk(m, _):
            pltpu.sync_copy(zbuf, acc_sh.at[pl.ds(base + m * _ZCH, _ZCH)])
            return 0
        lax.fori_loop(0, _NZ, zero_chunk, 0)
        plsc.subcore_barrier()

        rbase = s * _R2
        for o in range(_OUTER2):
            pltpu.sync_copy(dst_hbm.at[c, pl.ds(rbase + o * _KCH2, _KCH2), :],
                            didx)

            def inner(kk, _):
                pltpu.sync_copy(ones_v, acc_sh.at[didx.at[kk]], add=True)
                return 0
            lax.fori_loop(0, _KCH2, inner, 0)

        plsc.subcore_barrier()

        def copy_out(m, _):
            off = base + m * _ZCH
            pltpu.sync_copy(acc_sh.at[pl.ds(off, _ZCH)], zbuf)
            pltpu.sync_copy(zbuf, out_hbm.at[c, pl.ds(off, _ZCH), :])
            return 0
        lax.fori_loop(0, _NZ, copy_out, 0)

    return k(dst2)


_NBUF = 8
_LA = 4
_NGRP = _KCH // _NBUF


def _sc_edge(hwp, src2, dstr):
    @functools.partial(
        pl.kernel,
        out_type=jax.ShapeDtypeStruct((_NC, _NP, _LANE), F32),
        mesh=_sc_mesh(),
        compiler_params=pltpu.CompilerParams(use_tc_tiling_on_sc=False),
        scratch_types=[
            pltpu.VMEM((_KCH, 128), jnp.int32),
            pltpu.VMEM((_KCH, 128), jnp.int32),
            pltpu.VMEM((_NBUF, 128, _LANE), F32),
            pltpu.VMEM_SHARED((_NP, _LANE), F32),
            pltpu.SemaphoreType.DMA((_NBUF,)),
            pltpu.SemaphoreType.DMA((_NBUF,)),
        ],
    )
    def k(hwp_hbm, src_hbm, dst_hbm, out_hbm, sidx, didx, rows, acc_sh,
          gsem, ssem):
        c = lax.axis_index("c")
        s = lax.axis_index("s")
        _fill_rows(rows.at[0], _ZCH, 0.0)
        base = s * _NPT

        def zero_chunk(m, _):
            pltpu.sync_copy(rows.at[0],
                            acc_sh.at[pl.ds(base + m * _ZCH, _ZCH)])
            return 0
        lax.fori_loop(0, _NZ, zero_chunk, 0)
        plsc.subcore_barrier()

        def gather(j, b):
            pltpu.async_copy(hwp_hbm.at[sidx.at[j]], rows.at[b], gsem.at[b])

        def gather_wait(j, b):
            pltpu.make_async_copy(hwp_hbm.at[sidx.at[j]], rows.at[b],
                                  gsem.at[b]).wait()

        def scat(j, b):
            pltpu.async_copy(rows.at[b], acc_sh.at[didx.at[j]], ssem.at[b],
                             add=True)

        def scat_wait(j, b):
            pltpu.make_async_copy(rows.at[b], acc_sh.at[didx.at[j]],
                                  ssem.at[b]).wait()

        rbase = s * _R
        for o in range(_OUTER):
            if o > 0:
                for u in range(_NBUF):
                    scat_wait(0, u)
            pltpu.sync_copy(src_hbm.at[c, pl.ds(rbase + o * _KCH, _KCH), :],
                            sidx)
            pltpu.sync_copy(dst_hbm.at[pl.ds(rbase + o * _KCH, _KCH), :],
                            didx)

            for b in range(_LA):
                gather(b, b)

            for u in range(_NBUF):
                gather_wait(u, u)
                scat(u, u)
                bb = (u + _LA) % _NBUF
                if u >= _NBUF - _LA:
                    scat_wait(0, bb)
                gather(u + _LA, bb)

            def group(g, _):
                for u in range(_NBUF):
                    j = g * _NBUF + u
                    gather_wait(j, u)
                    scat(j, u)
                    bb = (u + _LA) % _NBUF
                    scat_wait(0, bb)
                    gather(j + _LA, bb)
                return 0
            lax.fori_loop(1, _NGRP - 1, group, 0)

            for u in range(_NBUF):
                j = (_NGRP - 1) * _NBUF + u
                gather_wait(j, u)
                scat(j, u)
                if u < _NBUF - _LA:
                    bb = (u + _LA) % _NBUF
                    scat_wait(0, bb)
                    gather(j + _LA, bb)

        for u in range(_NBUF):
            scat_wait(0, u)

        plsc.subcore_barrier()

        def copy_out(m, _):
            off = base + m * _ZCH
            pltpu.sync_copy(acc_sh.at[pl.ds(off, _ZCH)], rows.at[0])
            pltpu.sync_copy(rows.at[0], out_hbm.at[c, pl.ds(off, _ZCH), :])
            return 0
        lax.fori_loop(0, _NZ, copy_out, 0)

    return k(hwp, src2, dstr)


_NPB = _NP // 8
_BLKP = _BLK // 8
_FLAV = 17


def _tc_a_body(x_ref, deg_ref, emb_ref, w_ref, dinv_ref, hwp_ref):
    i = pl.program_id(0)
    dp = deg_ref[0] + deg_ref[1]
    row_iota = lax.broadcasted_iota(jnp.int32, (_BLKP, 128), 0)
    lane_iota = lax.broadcasted_iota(jnp.int32, (_BLKP, 128), 1)
    nid = 8 * (i * _BLKP + row_iota) + lane_iota // _LANE
    dinv = jnp.where(nid < _N, lax.rsqrt(dp + 1.0), 0.0)
    ew = jnp.dot(emb_ref[:], w_ref[:], preferred_element_type=F32)
    ew0 = jnp.concatenate([ew[:, :_LANE]] * 8, axis=1)
    ew1 = jnp.concatenate([ew[:, _LANE:]] * 8, axis=1)
    xq = x_ref[:]
    h0 = jnp.zeros((_BLKP, 128), F32)
    h1 = jnp.zeros((_BLKP, 128), F32)
    for f in range(_FLAV):
        sel = xq == f
        h0 = jnp.where(sel, ew0[f:f + 1, :], h0)
        h1 = jnp.where(sel, ew1[f:f + 1, :], h1)
    dinv_ref[:] = dinv
    hwp_ref[0, :, :] = dinv * h0
    hwp_ref[1, :, :] = dinv * h1


def _tc_a(xpk, degp, emb_pad, w0):
    return pl.pallas_call(
        _tc_a_body,
        grid=(_GRID,),
        in_specs=[
            pl.BlockSpec((_BLKP, 128), lambda i: (i, 0)),
            pl.BlockSpec((_NC, _BLKP, 128), lambda i: (0, i, 0)),
            pl.BlockSpec((_C, _C), lambda i: (0, 0)),
            pl.BlockSpec((_C, _C), lambda i: (0, 0)),
        ],
        out_specs=[
            pl.BlockSpec((_BLKP, 128), lambda i: (i, 0)),
            pl.BlockSpec((_NC, _BLKP, 128), lambda i: (0, i, 0)),
        ],
        out_shape=[
            jax.ShapeDtypeStruct((_NPB, 128), F32),
            jax.ShapeDtypeStruct((_NC, _NPB, 128), F32),
        ],
    )(xpk, degp, emb_pad, w0)


def _layer_h(acc_ref, hwp_ref, dinv_ref, b_ref):
    dinv = dinv_ref[:]
    h0 = jnp.maximum(dinv * (acc_ref[0] + hwp_ref[0]) + b_ref[0:1, :], 0.0)
    h1 = jnp.maximum(dinv * (acc_ref[1] + hwp_ref[1]) + b_ref[1:2, :], 0.0)
    return dinv, h0, h1


def _tc_b_body(acc_ref, hwp_ref, dinv_ref, b_ref, wk_ref, out_ref):
    dinv, h0, h1 = _layer_h(acc_ref, hwp_ref, dinv_ref, b_ref)
    hw0 = (jnp.dot(h0, wk_ref[0], preferred_element_type=F32)
           + jnp.dot(h1, wk_ref[2], preferred_element_type=F32))
    hw1 = (jnp.dot(h0, wk_ref[1], preferred_element_type=F32)
           + jnp.dot(h1, wk_ref[3], preferred_element_type=F32))
    out_ref[0, :, :] = dinv * hw0
    out_ref[1, :, :] = dinv * hw1


def _tc_b(acc, hwp, dinvp, bt, wk):
    return pl.pallas_call(
        _tc_b_body,
        grid=(_GRID,),
        in_specs=[
            pl.BlockSpec((_NC, _BLKP, 128), lambda i: (0, i, 0)),
            pl.BlockSpec((_NC, _BLKP, 128), lambda i: (0, i, 0)),
            pl.BlockSpec((_BLKP, 128), lambda i: (i, 0)),
            pl.BlockSpec((2, 128), lambda i: (0, 0)),
            pl.BlockSpec((4, 128, 128), lambda i: (0, 0, 0)),
        ],
        out_specs=pl.BlockSpec((_NC, _BLKP, 128), lambda i: (0, i, 0)),
        out_shape=jax.ShapeDtypeStruct((_NC, _NPB, 128), F32),
    )(acc, hwp, dinvp, bt, wk)


def _tc_seg_body(acc_ref, hwp_ref, dinv_ref, bat_ref, b_ref, d0w_ref,
                 d0b_ref, dw_ref, db_ref, fw_ref, fb_ref, out_ref, smax_ref):
    i = pl.program_id(0)

    @pl.when(i == 0)
    def _():
        smax_ref[:] = jnp.full((_G + 8, _C), NEG_INF, F32)

    _, h0, h1 = _layer_h(acc_ref, hwp_ref, dinv_ref, b_ref)

    bi = bat_ref[:]
    g_first = bat_ref[0, 0]
    g_last = bat_ref[_BLKP - 1, 127]

    def upd(g, _):
        m0 = jnp.max(jnp.where(bi == g, h0, NEG_INF), axis=0, keepdims=True)
        m1 = jnp.max(jnp.where(bi == g, h1, NEG_INF), axis=0, keepdims=True)
        r0 = m0[:, 0:_LANE]
        r1 = m1[:, 0:_LANE]
        for k in range(1, 8):
            r0 = jnp.maximum(r0, m0[:, k * _LANE:(k + 1) * _LANE])
            r1 = jnp.maximum(r1, m1[:, k * _LANE:(k + 1) * _LANE])
        m = jnp.concatenate([r0, r1], axis=1)
        cur = smax_ref[pl.ds(g, 1), :]
        smax_ref[pl.ds(g, 1), :] = jnp.maximum(cur, m)
        return 0
    lax.fori_loop(g_first, g_last + 1, upd, 0)

    @pl.when(i == _GRID - 1)
    def _():
        g = smax_ref[0:_G, :]
        g = jnp.maximum(
            jnp.dot(g, d0w_ref[:], preferred_element_type=F32) + d0b_ref[:],
            0.0)
        for j in range(_DENSE_LAYERS):
            g = jnp.maximum(
                jnp.dot(g, dw_ref[j], preferred_element_type=F32)
                + db_ref[j], 0.0)
        logits = jnp.dot(g, fw_ref[:], preferred_element_type=F32) + fb_ref[:]
        m = jnp.max(logits, axis=1, keepdims=True)
        z = logits - m
        lse = jnp.log(jnp.sum(jnp.exp(z), axis=1, keepdims=True))
        out_ref[:] = (z - lse)[:, 0:2]


def _tc_seg(acc, hwp, dinvp, batpk, bt, d0w, d0b, dw, db, fw, fb):
    return pl.pallas_call(
        _tc_seg_body,
        grid=(_GRID,),
        in_specs=[
            pl.BlockSpec((_NC, _BLKP, 128), lambda i: (0, i, 0)),
            pl.BlockSpec((_NC, _BLKP, 128), lambda i: (0, i, 0)),
            pl.BlockSpec((_BLKP, 128), lambda i: (i, 0)),
            pl.BlockSpec((_BLKP, 128), lambda i: (i, 0)),
            pl.BlockSpec((2, 128), lambda i: (0, 0)),
            pl.BlockSpec((_C, _C), lambda i: (0, 0)),
            pl.BlockSpec((1, _C), lambda i: (0, 0)),
            pl.BlockSpec((_DENSE_LAYERS, _C, _C), lambda i: (0, 0, 0)),
            pl.BlockSpec((_DENSE_LAYERS, 1, _C), lambda i: (0, 0, 0)),
            pl.BlockSpec((_C, 8), lambda i: (0, 0)),
            pl.BlockSpec((1, 8), lambda i: (0, 0)),
        ],
        out_specs=pl.BlockSpec((_G, 2), lambda i: (0, 0)),
        out_shape=jax.ShapeDtypeStruct((_G, 2), F32),
        scratch_shapes=[pltpu.VMEM((_G + 8, _C), F32)],
    )(acc, hwp, dinvp, batpk, bt, d0w, d0b, dw, db, fw, fb)


def _pack_scalar(v, pad_value):
    vp = jnp.pad(v, (0, _NP - _N), constant_values=pad_value)
    return jnp.repeat(vp, _LANE).reshape(_NPB, 128)


def kernel(x, edge_index, batch, embed, conv_W, conv_b, dense0_W, dense0_b,
           dense_W, dense_b, final_W, final_b):
    x32 = x.astype(jnp.int32)
    src = edge_index[0].astype(jnp.int32)
    dst = edge_index[1].astype(jnp.int32)
    bat = batch.astype(jnp.int32)

    xpk = _pack_scalar(x32, 0)
    batpk = _pack_scalar(bat, _G)

    src_pad = jnp.pad(src, (0, _EP - _E), constant_values=_N)
    src2 = jnp.stack([src_pad, src_pad + _NP]).reshape(_NC, _RT, 128)
    dstr = jnp.pad(dst, (0, _EP - _E),
                   constant_values=_N).reshape(_RT, 128)

    dh0 = jnp.pad(dst[:_EH], (0, _EP2 - _EH), constant_values=_N)
    dh1 = jnp.pad(dst[_EH:], (0, _EP2 - (_E - _EH)), constant_values=_N)
    dst2 = jnp.stack([dh0, dh1]).reshape(_NC, _RT2, 128)

    emb_pad = jnp.zeros((_C, _C), F32).at[:embed.shape[0]].set(embed)

    eye8 = jnp.eye(8, dtype=F32)
    wks = []
    for l in range(1, _CONV_LAYERS):
        w = conv_W[l]
        wks.append(jnp.stack([
            jnp.kron(eye8, w[:_LANE, :_LANE]),
            jnp.kron(eye8, w[:_LANE, _LANE:]),
            jnp.kron(eye8, w[_LANE:, :_LANE]),
            jnp.kron(eye8, w[_LANE:, _LANE:]),
        ]))
    bts = [jnp.stack([jnp.tile(conv_b[l][:_LANE], 8),
                      jnp.tile(conv_b[l][_LANE:], 8)])
           for l in range(_CONV_LAYERS)]

    d0b = dense0_b.reshape(1, _C)
    db = dense_b.reshape(_DENSE_LAYERS, 1, _C)
    fw = jnp.zeros((_C, 8), F32).at[:, :2].set(final_W)
    fb = jnp.full((1, 8), -1e30, F32).at[0, :2].set(final_b)

    degp = _sc_deg(dst2)
    dinvp, hwp = _tc_a(xpk, degp.reshape(_NC, _NPB, 128), emb_pad, conv_W[0])

    for l in range(_CONV_LAYERS):
        acc = _sc_edge(hwp.reshape(_NC * _NP, _LANE), src2, dstr)
        accp = acc.reshape(_NC, _NPB, 128)
        if l + 1 < _CONV_LAYERS:
            hwp = _tc_b(accp, hwp, dinvp, bts[l], wks[l])
        else:
            out = _tc_seg(accp, hwp, dinvp, batpk, bts[l], dense0_W,
                          d0b, dense_W, db, fw, fb)
    return out


# speedup vs baseline: 39.1206x; 1.0324x over previous
import functools

import jax
import jax.numpy as jnp
from jax import lax
from jax.experimental import pallas as pl
from jax.experimental.pallas import tpu as pltpu
from jax.experimental.pallas import tpu_sc as plsc

F32 = jnp.float32
NEG_INF = float("-inf")

_N = 100000
_E = 1600000
_C = 32
_G = 64
_CONV_LAYERS = 5
_DENSE_LAYERS = 3

_NC = 2
_NS = 16
_LANE = 16

_BLK = 2048
_NP = 100352
_GRID = _NP // _BLK
_NPT = _NP // _NS
_ZCH = 128
_NZ = _NPT // _ZCH

_KCH = 56
_R = 784
_OUTER = _R // _KCH
_RT = _NS * _R
_EP = _RT * 128

_EH = _E // 2
_KCH2 = 56
_R2 = 392
_OUTER2 = _R2 // _KCH2
_RT2 = _NS * _R2
_EP2 = _RT2 * 128


def _fill_rows(ref, nrows, value):
    def body(i, _):
        ref[i, :] = jnp.full((_LANE,), value, F32)
        return 0
    lax.fori_loop(0, nrows, body, 0)


def _sc_mesh():
    return plsc.VectorSubcoreMesh(core_axis_name="c", subcore_axis_name="s")


def _sc_deg(dstr):
    @functools.partial(
        pl.kernel,
        out_type=jax.ShapeDtypeStruct((_NC, _NP, _LANE), F32),
        mesh=_sc_mesh(),
        compiler_params=pltpu.CompilerParams(use_tc_tiling_on_sc=False),
        scratch_types=[
            pltpu.VMEM((_KCH2, 128), jnp.int32),
            pltpu.VMEM((128, _LANE), F32),
            pltpu.VMEM((_ZCH, _LANE), F32),
            pltpu.VMEM_SHARED((_NP, _LANE), F32),
        ],
    )
    def k(dst_hbm, out_hbm, didx, ones_v, zbuf, acc_sh):
        c = lax.axis_index("c")
        s = lax.axis_index("s")
        _fill_rows(zbuf, _ZCH, 0.0)
        _fill_rows(ones_v, 128, 1.0)
        base = s * _NPT

        def zero_chunk(m, _):
            pltpu.sync_copy(zbuf, acc_sh.at[pl.ds(base + m * _ZCH, _ZCH)])
            return 0
        lax.fori_loop(0, _NZ, zero_chunk, 0)
        plsc.subcore_barrier()

        rbase = c * (_RT // 2) + s * _R2
        for o in range(_OUTER2):
            pltpu.sync_copy(dst_hbm.at[pl.ds(rbase + o * _KCH2, _KCH2), :],
                            didx)

            def inner(kk, _):
                pltpu.sync_copy(ones_v, acc_sh.at[didx.at[kk]], add=True)
                return 0
            lax.fori_loop(0, _KCH2, inner, 0)

        plsc.subcore_barrier()

        def copy_out(m, _):
            off = base + m * _ZCH
            pltpu.sync_copy(acc_sh.at[pl.ds(off, _ZCH)], zbuf)
            pltpu.sync_copy(zbuf, out_hbm.at[c, pl.ds(off, _ZCH), :])
            return 0
        lax.fori_loop(0, _NZ, copy_out, 0)

    return k(dstr)


_NBUF = 8
_LA = 4
_NGRP = _KCH // _NBUF


def _sc_edge(hwp, srcr, dstr):
    @functools.partial(
        pl.kernel,
        out_type=jax.ShapeDtypeStruct((_NC, _NP, _LANE), F32),
        mesh=_sc_mesh(),
        compiler_params=pltpu.CompilerParams(use_tc_tiling_on_sc=False),
        scratch_types=[
            pltpu.VMEM((_KCH, 128), jnp.int32),
            pltpu.VMEM((_KCH, 128), jnp.int32),
            pltpu.VMEM((_NBUF, 128, _LANE), F32),
            pltpu.VMEM_SHARED((_NP, _LANE), F32),
            pltpu.SemaphoreType.DMA((_NBUF,)),
            pltpu.SemaphoreType.DMA((_NBUF,)),
        ],
    )
    def k(hwp_hbm, src_hbm, dst_hbm, out_hbm, sidx, didx, rows, acc_sh,
          gsem, ssem):
        c = lax.axis_index("c")
        s = lax.axis_index("s")
        _fill_rows(rows.at[0], _ZCH, 0.0)
        base = s * _NPT

        def zero_chunk(m, _):
            pltpu.sync_copy(rows.at[0],
                            acc_sh.at[pl.ds(base + m * _ZCH, _ZCH)])
            return 0
        lax.fori_loop(0, _NZ, zero_chunk, 0)
        plsc.subcore_barrier()

        def gather(j, b):
            pltpu.async_copy(hwp_hbm.at[c].at[sidx.at[j]], rows.at[b],
                             gsem.at[b])

        def gather_wait(j, b):
            pltpu.make_async_copy(hwp_hbm.at[c].at[sidx.at[j]], rows.at[b],
                                  gsem.at[b]).wait()

        def scat(j, b):
            pltpu.async_copy(rows.at[b], acc_sh.at[didx.at[j]], ssem.at[b],
                             add=True)

        def scat_wait(j, b):
            pltpu.make_async_copy(rows.at[b], acc_sh.at[didx.at[j]],
                                  ssem.at[b]).wait()

        rbase = s * _R
        for o in range(_OUTER):
            if o > 0:
                for u in range(_NBUF):
                    scat_wait(0, u)
            pltpu.sync_copy(src_hbm.at[pl.ds(rbase + o * _KCH, _KCH), :],
                            sidx)
            pltpu.sync_copy(dst_hbm.at[pl.ds(rbase + o * _KCH, _KCH), :],
                            didx)

            for b in range(_LA):
                gather(b, b)

            for u in range(_NBUF):
                gather_wait(u, u)
                scat(u, u)
                bb = (u + _LA) % _NBUF
                if u >= _NBUF - _LA:
                    scat_wait(0, bb)
                gather(u + _LA, bb)

            def group(g, _):
                for u in range(_NBUF):
                    j = g * _NBUF + u
                    gather_wait(j, u)
                    scat(j, u)
                    bb = (u + _LA) % _NBUF
                    scat_wait(0, bb)
                    gather(j + _LA, bb)
                return 0
            lax.fori_loop(1, _NGRP - 1, group, 0)

            for u in range(_NBUF):
                j = (_NGRP - 1) * _NBUF + u
                gather_wait(j, u)
                scat(j, u)
                if u < _NBUF - _LA:
                    bb = (u + _LA) % _NBUF
                    scat_wait(0, bb)
                    gather(j + _LA, bb)

        for u in range(_NBUF):
            scat_wait(0, u)

        plsc.subcore_barrier()

        def copy_out(m, _):
            off = base + m * _ZCH
            pltpu.sync_copy(acc_sh.at[pl.ds(off, _ZCH)], rows.at[0])
            pltpu.sync_copy(rows.at[0], out_hbm.at[c, pl.ds(off, _ZCH), :])
            return 0
        lax.fori_loop(0, _NZ, copy_out, 0)

    return k(hwp, srcr, dstr)


_NPB = _NP // 8
_BLKP = _BLK // 8
_FLAV = 17


def _tc_a_body(x_ref, deg_ref, emb_ref, w_ref, dinv_ref, hwp_ref):
    i = pl.program_id(0)
    dp = deg_ref[0] + deg_ref[1]
    row_iota = lax.broadcasted_iota(jnp.int32, (_BLKP, 128), 0)
    lane_iota = lax.broadcasted_iota(jnp.int32, (_BLKP, 128), 1)
    nid = 8 * (i * _BLKP + row_iota) + lane_iota // _LANE
    dinv = jnp.where(nid < _N, lax.rsqrt(dp + 1.0), 0.0)
    ew = jnp.dot(emb_ref[:], w_ref[:], preferred_element_type=F32)
    ew0 = jnp.concatenate([ew[:, :_LANE]] * 8, axis=1)
    ew1 = jnp.concatenate([ew[:, _LANE:]] * 8, axis=1)
    xq = x_ref[:]
    h0 = jnp.zeros((_BLKP, 128), F32)
    h1 = jnp.zeros((_BLKP, 128), F32)
    for f in range(_FLAV):
        sel = xq == f
        h0 = jnp.where(sel, ew0[f:f + 1, :], h0)
        h1 = jnp.where(sel, ew1[f:f + 1, :], h1)
    dinv_ref[:] = dinv
    hwp_ref[0, :, :] = dinv * h0
    hwp_ref[1, :, :] = dinv * h1


def _tc_a(xpk, degp, emb_pad, w0):
    return pl.pallas_call(
        _tc_a_body,
        grid=(_GRID,),
        in_specs=[
            pl.BlockSpec((_BLKP, 128), lambda i: (i, 0)),
            pl.BlockSpec((_NC, _BLKP, 128), lambda i: (0, i, 0)),
            pl.BlockSpec((_C, _C), lambda i: (0, 0)),
            pl.BlockSpec((_C, _C), lambda i: (0, 0)),
        ],
        out_specs=[
            pl.BlockSpec((_BLKP, 128), lambda i: (i, 0)),
            pl.BlockSpec((_NC, _BLKP, 128), lambda i: (0, i, 0)),
        ],
        out_shape=[
            jax.ShapeDtypeStruct((_NPB, 128), F32),
            jax.ShapeDtypeStruct((_NC, _NPB, 128), F32),
        ],
    )(xpk, degp, emb_pad, w0)


def _layer_h(acc_ref, hwp_ref, dinv_ref, b_ref):
    dinv = dinv_ref[:]
    h0 = jnp.maximum(dinv * (acc_ref[0] + hwp_ref[0]) + b_ref[0:1, :], 0.0)
    h1 = jnp.maximum(dinv * (acc_ref[1] + hwp_ref[1]) + b_ref[1:2, :], 0.0)
    return dinv, h0, h1


def _tc_b_body(acc_ref, hwp_ref, dinv_ref, b_ref, wk_ref, out_ref):
    dinv, h0, h1 = _layer_h(acc_ref, hwp_ref, dinv_ref, b_ref)
    hw0 = (jnp.dot(h0, wk_ref[0], preferred_element_type=F32)
           + jnp.dot(h1, wk_ref[2], preferred_element_type=F32))
    hw1 = (jnp.dot(h0, wk_ref[1], preferred_element_type=F32)
           + jnp.dot(h1, wk_ref[3], preferred_element_type=F32))
    out_ref[0, :, :] = dinv * hw0
    out_ref[1, :, :] = dinv * hw1


def _tc_b(acc, hwp, dinvp, bt, wk):
    return pl.pallas_call(
        _tc_b_body,
        grid=(_GRID,),
        in_specs=[
            pl.BlockSpec((_NC, _BLKP, 128), lambda i: (0, i, 0)),
            pl.BlockSpec((_NC, _BLKP, 128), lambda i: (0, i, 0)),
            pl.BlockSpec((_BLKP, 128), lambda i: (i, 0)),
            pl.BlockSpec((2, 128), lambda i: (0, 0)),
            pl.BlockSpec((4, 128, 128), lambda i: (0, 0, 0)),
        ],
        out_specs=pl.BlockSpec((_NC, _BLKP, 128), lambda i: (0, i, 0)),
        out_shape=jax.ShapeDtypeStruct((_NC, _NPB, 128), F32),
    )(acc, hwp, dinvp, bt, wk)


def _tc_seg_body(acc_ref, hwp_ref, dinv_ref, bat_ref, b_ref, d0w_ref,
                 d0b_ref, dw_ref, db_ref, fw_ref, fb_ref, out_ref, smax_ref):
    i = pl.program_id(0)

    @pl.when(i == 0)
    def _():
        smax_ref[:] = jnp.full((_G + 8, _C), NEG_INF, F32)

    _, h0, h1 = _layer_h(acc_ref, hwp_ref, dinv_ref, b_ref)

    bi = bat_ref[:]
    g_first = bat_ref[0, 0]
    g_last = bat_ref[_BLKP - 1, 127]

    def upd(g, _):
        m0 = jnp.max(jnp.where(bi == g, h0, NEG_INF), axis=0, keepdims=True)
        m1 = jnp.max(jnp.where(bi == g, h1, NEG_INF), axis=0, keepdims=True)
        r0 = m0[:, 0:_LANE]
        r1 = m1[:, 0:_LANE]
        for k in range(1, 8):
            r0 = jnp.maximum(r0, m0[:, k * _LANE:(k + 1) * _LANE])
            r1 = jnp.maximum(r1, m1[:, k * _LANE:(k + 1) * _LANE])
        m = jnp.concatenate([r0, r1], axis=1)
        cur = smax_ref[pl.ds(g, 1), :]
        smax_ref[pl.ds(g, 1), :] = jnp.maximum(cur, m)
        return 0
    lax.fori_loop(g_first, g_last + 1, upd, 0)

    @pl.when(i == _GRID - 1)
    def _():
        g = smax_ref[0:_G, :]
        g = jnp.maximum(
            jnp.dot(g, d0w_ref[:], preferred_element_type=F32) + d0b_ref[:],
            0.0)
        for j in range(_DENSE_LAYERS):
            g = jnp.maximum(
                jnp.dot(g, dw_ref[j], preferred_element_type=F32)
                + db_ref[j], 0.0)
        logits = jnp.dot(g, fw_ref[:], preferred_element_type=F32) + fb_ref[:]
        m = jnp.max(logits, axis=1, keepdims=True)
        z = logits - m
        lse = jnp.log(jnp.sum(jnp.exp(z), axis=1, keepdims=True))
        out_ref[:] = (z - lse)[:, 0:2]


def _tc_seg(acc, hwp, dinvp, batpk, bt, d0w, d0b, dw, db, fw, fb):
    return pl.pallas_call(
        _tc_seg_body,
        grid=(_GRID,),
        in_specs=[
            pl.BlockSpec((_NC, _BLKP, 128), lambda i: (0, i, 0)),
            pl.BlockSpec((_NC, _BLKP, 128), lambda i: (0, i, 0)),
            pl.BlockSpec((_BLKP, 128), lambda i: (i, 0)),
            pl.BlockSpec((_BLKP, 128), lambda i: (i, 0)),
            pl.BlockSpec((2, 128), lambda i: (0, 0)),
            pl.BlockSpec((_C, _C), lambda i: (0, 0)),
            pl.BlockSpec((1, _C), lambda i: (0, 0)),
            pl.BlockSpec((_DENSE_LAYERS, _C, _C), lambda i: (0, 0, 0)),
            pl.BlockSpec((_DENSE_LAYERS, 1, _C), lambda i: (0, 0, 0)),
            pl.BlockSpec((_C, 8), lambda i: (0, 0)),
            pl.BlockSpec((1, 8), lambda i: (0, 0)),
        ],
        out_specs=pl.BlockSpec((_G, 2), lambda i: (0, 0)),
        out_shape=jax.ShapeDtypeStruct((_G, 2), F32),
        scratch_shapes=[pltpu.VMEM((_G + 8, _C), F32)],
    )(acc, hwp, dinvp, batpk, bt, d0w, d0b, dw, db, fw, fb)


def _pack_scalar(v, pad_value):
    vp = jnp.pad(v, (0, _NP - _N), constant_values=pad_value)
    return jnp.repeat(vp, _LANE).reshape(_NPB, 128)


def kernel(x, edge_index, batch, embed, conv_W, conv_b, dense0_W, dense0_b,
           dense_W, dense_b, final_W, final_b):
    x32 = x.astype(jnp.int32)
    src = edge_index[0].astype(jnp.int32)
    dst = edge_index[1].astype(jnp.int32)
    bat = batch.astype(jnp.int32)

    xpk = _pack_scalar(x32, 0)
    batpk = _pack_scalar(bat, _G)

    srcr = jnp.pad(src, (0, _EP - _E), constant_values=_N).reshape(_RT, 128)
    dstr = jnp.pad(dst, (0, _EP - _E),
                   constant_values=_N).reshape(_RT, 128)

    emb_pad = jnp.zeros((_C, _C), F32).at[:embed.shape[0]].set(embed)

    eye8 = jnp.eye(8, dtype=F32)
    wks = []
    for l in range(1, _CONV_LAYERS):
        w = conv_W[l]
        wks.append(jnp.stack([
            jnp.kron(eye8, w[:_LANE, :_LANE]),
            jnp.kron(eye8, w[:_LANE, _LANE:]),
            jnp.kron(eye8, w[_LANE:, :_LANE]),
            jnp.kron(eye8, w[_LANE:, _LANE:]),
        ]))
    bts = [jnp.stack([jnp.tile(conv_b[l][:_LANE], 8),
                      jnp.tile(conv_b[l][_LANE:], 8)])
           for l in range(_CONV_LAYERS)]

    d0b = dense0_b.reshape(1, _C)
    db = dense_b.reshape(_DENSE_LAYERS, 1, _C)
    fw = jnp.zeros((_C, 8), F32).at[:, :2].set(final_W)
    fb = jnp.full((1, 8), -1e30, F32).at[0, :2].set(final_b)

    degp = _sc_deg(dstr)
    dinvp, hwp = _tc_a(xpk, degp.reshape(_NC, _NPB, 128), emb_pad, conv_W[0])

    for l in range(_CONV_LAYERS):
        acc = _sc_edge(hwp.reshape(_NC, _NP, _LANE), srcr, dstr)
        accp = acc.reshape(_NC, _NPB, 128)
        if l + 1 < _CONV_LAYERS:
            hwp = _tc_b(accp, hwp, dinvp, bts[l], wks[l])
        else:
            out = _tc_seg(accp, hwp, dinvp, batpk, bts[l], dense0_W,
                          d0b, dense_W, db, fw, fb)
    return out


# speedup vs baseline: 39.3900x; 1.0069x over previous
import functools

import jax
import jax.numpy as jnp
from jax import lax
from jax.experimental import pallas as pl
from jax.experimental.pallas import tpu as pltpu
from jax.experimental.pallas import tpu_sc as plsc

F32 = jnp.float32
NEG_INF = float("-inf")

_N = 100000
_E = 1600000
_C = 32
_G = 64
_CONV_LAYERS = 5
_DENSE_LAYERS = 3

_NC = 2
_NS = 16
_LANE = 16

_BLK = 2048
_NP = 100352
_GRID = _NP // _BLK
_NPT = _NP // _NS
_ZCH = 128
_NZ = _NPT // _ZCH

_KCH = 56
_R = 784
_OUTER = _R // _KCH
_RT = _NS * _R
_EP = _RT * 128

_EH = _E // 2
_KCH2 = 56
_R2 = 392
_OUTER2 = _R2 // _KCH2
_RT2 = _NS * _R2
_EP2 = _RT2 * 128


def _fill_rows(ref, nrows, value):
    def body(i, _):
        ref[i, :] = jnp.full((_LANE,), value, F32)
        return 0
    lax.fori_loop(0, nrows, body, 0)


def _sc_mesh():
    return plsc.VectorSubcoreMesh(core_axis_name="c", subcore_axis_name="s")


def _sc_deg(dstr):
    @functools.partial(
        pl.kernel,
        out_type=jax.ShapeDtypeStruct((_NC, _NP, _LANE), F32),
        mesh=_sc_mesh(),
        compiler_params=pltpu.CompilerParams(use_tc_tiling_on_sc=False),
        scratch_types=[
            pltpu.VMEM((_KCH2, 128), jnp.int32),
            pltpu.VMEM((128, _LANE), F32),
            pltpu.VMEM((_ZCH, _LANE), F32),
            pltpu.VMEM_SHARED((_NP, _LANE), F32),
            pltpu.SemaphoreType.DMA,
        ],
    )
    def k(dst_hbm, out_hbm, didx, ones_v, zbuf, acc_sh, dsem):
        c = lax.axis_index("c")
        s = lax.axis_index("s")
        _fill_rows(zbuf, _ZCH, 0.0)
        _fill_rows(ones_v, 128, 1.0)
        base = s * _NPT

        def zero_chunk(m, _):
            pltpu.sync_copy(zbuf, acc_sh.at[pl.ds(base + m * _ZCH, _ZCH)])
            return 0
        lax.fori_loop(0, _NZ, zero_chunk, 0)
        plsc.subcore_barrier()

        rbase = c * (_RT // 2) + s * _R2
        for o in range(_OUTER2):
            pltpu.sync_copy(dst_hbm.at[pl.ds(rbase + o * _KCH2, _KCH2), :],
                            didx)

            def inner(kk, _):
                pltpu.async_copy(ones_v, acc_sh.at[didx.at[kk]], dsem,
                                 add=True)
                return 0
            lax.fori_loop(0, _KCH2, inner, 0)

            def drain(kk, _):
                pltpu.make_async_copy(ones_v, acc_sh.at[didx.at[0]],
                                      dsem).wait()
                return 0
            lax.fori_loop(0, _KCH2, drain, 0)

        plsc.subcore_barrier()

        def copy_out(m, _):
            off = base + m * _ZCH
            pltpu.sync_copy(acc_sh.at[pl.ds(off, _ZCH)], zbuf)
            pltpu.sync_copy(zbuf, out_hbm.at[c, pl.ds(off, _ZCH), :])
            return 0
        lax.fori_loop(0, _NZ, copy_out, 0)

    return k(dstr)


_NBUF = 8
_LA = 4
_NGRP = _KCH // _NBUF


def _sc_edge(hwp, srcr, dstr):
    @functools.partial(
        pl.kernel,
        out_type=jax.ShapeDtypeStruct((_NC, _NP, _LANE), F32),
        mesh=_sc_mesh(),
        compiler_params=pltpu.CompilerParams(use_tc_tiling_on_sc=False),
        scratch_types=[
            pltpu.VMEM((_KCH, 128), jnp.int32),
            pltpu.VMEM((_KCH, 128), jnp.int32),
            pltpu.VMEM((_NBUF, 128, _LANE), F32),
            pltpu.VMEM_SHARED((_NP, _LANE), F32),
            pltpu.SemaphoreType.DMA((_NBUF,)),
            pltpu.SemaphoreType.DMA((_NBUF,)),
        ],
    )
    def k(hwp_hbm, src_hbm, dst_hbm, out_hbm, sidx, didx, rows, acc_sh,
          gsem, ssem):
        c = lax.axis_index("c")
        s = lax.axis_index("s")
        _fill_rows(rows.at[0], _ZCH, 0.0)
        base = s * _NPT

        def zero_chunk(m, _):
            pltpu.sync_copy(rows.at[0],
                            acc_sh.at[pl.ds(base + m * _ZCH, _ZCH)])
            return 0
        lax.fori_loop(0, _NZ, zero_chunk, 0)
        plsc.subcore_barrier()

        def gather(j, b):
            pltpu.async_copy(hwp_hbm.at[c].at[sidx.at[j]], rows.at[b],
                             gsem.at[b])

        def gather_wait(j, b):
            pltpu.make_async_copy(hwp_hbm.at[c].at[sidx.at[j]], rows.at[b],
                                  gsem.at[b]).wait()

        def scat(j, b):
            pltpu.async_copy(rows.at[b], acc_sh.at[didx.at[j]], ssem.at[b],
                             add=True)

        def scat_wait(j, b):
            pltpu.make_async_copy(rows.at[b], acc_sh.at[didx.at[j]],
                                  ssem.at[b]).wait()

        rbase = s * _R
        for o in range(_OUTER):
            if o > 0:
                for u in range(_NBUF):
                    scat_wait(0, u)
            pltpu.sync_copy(src_hbm.at[pl.ds(rbase + o * _KCH, _KCH), :],
                            sidx)
            pltpu.sync_copy(dst_hbm.at[pl.ds(rbase + o * _KCH, _KCH), :],
                            didx)

            for b in range(_LA):
                gather(b, b)

            for u in range(_NBUF):
                gather_wait(u, u)
                scat(u, u)
                bb = (u + _LA) % _NBUF
                if u >= _NBUF - _LA:
                    scat_wait(0, bb)
                gather(u + _LA, bb)

            def group(g, _):
                for u in range(_NBUF):
                    j = g * _NBUF + u
                    gather_wait(j, u)
                    scat(j, u)
                    bb = (u + _LA) % _NBUF
                    scat_wait(0, bb)
                    gather(j + _LA, bb)
                return 0
            lax.fori_loop(1, _NGRP - 1, group, 0)

            for u in range(_NBUF):
                j = (_NGRP - 1) * _NBUF + u
                gather_wait(j, u)
                scat(j, u)
                if u < _NBUF - _LA:
                    bb = (u + _LA) % _NBUF
                    scat_wait(0, bb)
                    gather(j + _LA, bb)

        for u in range(_NBUF):
            scat_wait(0, u)

        plsc.subcore_barrier()

        def copy_out(m, _):
            off = base + m * _ZCH
            pltpu.sync_copy(acc_sh.at[pl.ds(off, _ZCH)], rows.at[0])
            pltpu.sync_copy(rows.at[0], out_hbm.at[c, pl.ds(off, _ZCH), :])
            return 0
        lax.fori_loop(0, _NZ, copy_out, 0)

    return k(hwp, srcr, dstr)


_NPB = _NP // 8
_BLKP = _BLK // 8
_FLAV = 17


def _tc_a_body(x_ref, deg_ref, emb_ref, w_ref, dinv_ref, hwp_ref):
    i = pl.program_id(0)
    dp = deg_ref[0] + deg_ref[1]
    row_iota = lax.broadcasted_iota(jnp.int32, (_BLKP, 128), 0)
    lane_iota = lax.broadcasted_iota(jnp.int32, (_BLKP, 128), 1)
    nid = 8 * (i * _BLKP + row_iota) + lane_iota // _LANE
    dinv = jnp.where(nid < _N, lax.rsqrt(dp + 1.0), 0.0)
    ew = jnp.dot(emb_ref[:], w_ref[:], preferred_element_type=F32)
    ew0 = jnp.concatenate([ew[:, :_LANE]] * 8, axis=1)
    ew1 = jnp.concatenate([ew[:, _LANE:]] * 8, axis=1)
    xq = x_ref[:]
    h0 = jnp.zeros((_BLKP, 128), F32)
    h1 = jnp.zeros((_BLKP, 128), F32)
    for f in range(_FLAV):
        sel = xq == f
        h0 = jnp.where(sel, ew0[f:f + 1, :], h0)
        h1 = jnp.where(sel, ew1[f:f + 1, :], h1)
    dinv_ref[:] = dinv
    hwp_ref[0, :, :] = dinv * h0
    hwp_ref[1, :, :] = dinv * h1


def _tc_a(xpk, degp, emb_pad, w0):
    return pl.pallas_call(
        _tc_a_body,
        grid=(_GRID,),
        in_specs=[
            pl.BlockSpec((_BLKP, 128), lambda i: (i, 0)),
            pl.BlockSpec((_NC, _BLKP, 128), lambda i: (0, i, 0)),
            pl.BlockSpec((_C, _C), lambda i: (0, 0)),
            pl.BlockSpec((_C, _C), lambda i: (0, 0)),
        ],
        out_specs=[
            pl.BlockSpec((_BLKP, 128), lambda i: (i, 0)),
            pl.BlockSpec((_NC, _BLKP, 128), lambda i: (0, i, 0)),
        ],
        out_shape=[
            jax.ShapeDtypeStruct((_NPB, 128), F32),
            jax.ShapeDtypeStruct((_NC, _NPB, 128), F32),
        ],
    )(xpk, degp, emb_pad, w0)


def _layer_h(acc_ref, hwp_ref, dinv_ref, b_ref):
    dinv = dinv_ref[:]
    h0 = jnp.maximum(dinv * (acc_ref[0] + hwp_ref[0]) + b_ref[0:1, :], 0.0)
    h1 = jnp.maximum(dinv * (acc_ref[1] + hwp_ref[1]) + b_ref[1:2, :], 0.0)
    return dinv, h0, h1


def _tc_b_body(acc_ref, hwp_ref, dinv_ref, b_ref, wk_ref, out_ref):
    dinv, h0, h1 = _layer_h(acc_ref, hwp_ref, dinv_ref, b_ref)
    hw0 = (jnp.dot(h0, wk_ref[0], preferred_element_type=F32)
           + jnp.dot(h1, wk_ref[2], preferred_element_type=F32))
    hw1 = (jnp.dot(h0, wk_ref[1], preferred_element_type=F32)
           + jnp.dot(h1, wk_ref[3], preferred_element_type=F32))
    out_ref[0, :, :] = dinv * hw0
    out_ref[1, :, :] = dinv * hw1


def _tc_b(acc, hwp, dinvp, bt, wk):
    return pl.pallas_call(
        _tc_b_body,
        grid=(_GRID,),
        in_specs=[
            pl.BlockSpec((_NC, _BLKP, 128), lambda i: (0, i, 0)),
            pl.BlockSpec((_NC, _BLKP, 128), lambda i: (0, i, 0)),
            pl.BlockSpec((_BLKP, 128), lambda i: (i, 0)),
            pl.BlockSpec((2, 128), lambda i: (0, 0)),
            pl.BlockSpec((4, 128, 128), lambda i: (0, 0, 0)),
        ],
        out_specs=pl.BlockSpec((_NC, _BLKP, 128), lambda i: (0, i, 0)),
        out_shape=jax.ShapeDtypeStruct((_NC, _NPB, 128), F32),
    )(acc, hwp, dinvp, bt, wk)


def _tc_seg_body(acc_ref, hwp_ref, dinv_ref, bat_ref, b_ref, d0w_ref,
                 d0b_ref, dw_ref, db_ref, fw_ref, fb_ref, out_ref, smax_ref):
    i = pl.program_id(0)

    @pl.when(i == 0)
    def _():
        smax_ref[:] = jnp.full((_G + 8, _C), NEG_INF, F32)

    _, h0, h1 = _layer_h(acc_ref, hwp_ref, dinv_ref, b_ref)

    bi = bat_ref[:]
    g_first = bat_ref[0, 0]
    g_last = bat_ref[_BLKP - 1, 127]

    def upd(g, _):
        m0 = jnp.max(jnp.where(bi == g, h0, NEG_INF), axis=0, keepdims=True)
        m1 = jnp.max(jnp.where(bi == g, h1, NEG_INF), axis=0, keepdims=True)
        r0 = m0[:, 0:_LANE]
        r1 = m1[:, 0:_LANE]
        for k in range(1, 8):
            r0 = jnp.maximum(r0, m0[:, k * _LANE:(k + 1) * _LANE])
            r1 = jnp.maximum(r1, m1[:, k * _LANE:(k + 1) * _LANE])
        m = jnp.concatenate([r0, r1], axis=1)
        cur = smax_ref[pl.ds(g, 1), :]
        smax_ref[pl.ds(g, 1), :] = jnp.maximum(cur, m)
        return 0
    lax.fori_loop(g_first, g_last + 1, upd, 0)

    @pl.when(i == _GRID - 1)
    def _():
        g = smax_ref[0:_G, :]
        g = jnp.maximum(
            jnp.dot(g, d0w_ref[:], preferred_element_type=F32) + d0b_ref[:],
            0.0)
        for j in range(_DENSE_LAYERS):
            g = jnp.maximum(
                jnp.dot(g, dw_ref[j], preferred_element_type=F32)
                + db_ref[j], 0.0)
        logits = jnp.dot(g, fw_ref[:], preferred_element_type=F32) + fb_ref[:]
        m = jnp.max(logits, axis=1, keepdims=True)
        z = logits - m
        lse = jnp.log(jnp.sum(jnp.exp(z), axis=1, keepdims=True))
        out_ref[:] = (z - lse)[:, 0:2]


def _tc_seg(acc, hwp, dinvp, batpk, bt, d0w, d0b, dw, db, fw, fb):
    return pl.pallas_call(
        _tc_seg_body,
        grid=(_GRID,),
        in_specs=[
            pl.BlockSpec((_NC, _BLKP, 128), lambda i: (0, i, 0)),
            pl.BlockSpec((_NC, _BLKP, 128), lambda i: (0, i, 0)),
            pl.BlockSpec((_BLKP, 128), lambda i: (i, 0)),
            pl.BlockSpec((_BLKP, 128), lambda i: (i, 0)),
            pl.BlockSpec((2, 128), lambda i: (0, 0)),
            pl.BlockSpec((_C, _C), lambda i: (0, 0)),
            pl.BlockSpec((1, _C), lambda i: (0, 0)),
            pl.BlockSpec((_DENSE_LAYERS, _C, _C), lambda i: (0, 0, 0)),
            pl.BlockSpec((_DENSE_LAYERS, 1, _C), lambda i: (0, 0, 0)),
            pl.BlockSpec((_C, 8), lambda i: (0, 0)),
            pl.BlockSpec((1, 8), lambda i: (0, 0)),
        ],
        out_specs=pl.BlockSpec((_G, 2), lambda i: (0, 0)),
        out_shape=jax.ShapeDtypeStruct((_G, 2), F32),
        scratch_shapes=[pltpu.VMEM((_G + 8, _C), F32)],
    )(acc, hwp, dinvp, batpk, bt, d0w, d0b, dw, db, fw, fb)


def _pack_scalar(v, pad_value):
    vp = jnp.pad(v, (0, _NP - _N), constant_values=pad_value)
    return jnp.repeat(vp, _LANE).reshape(_NPB, 128)


def kernel(x, edge_index, batch, embed, conv_W, conv_b, dense0_W, dense0_b,
           dense_W, dense_b, final_W, final_b):
    x32 = x.astype(jnp.int32)
    src = edge_index[0].astype(jnp.int32)
    dst = edge_index[1].astype(jnp.int32)
    bat = batch.astype(jnp.int32)

    xpk = _pack_scalar(x32, 0)
    batpk = _pack_scalar(bat, _G)

    srcr = jnp.pad(src, (0, _EP - _E), constant_values=_N).reshape(_RT, 128)
    dstr = jnp.pad(dst, (0, _EP - _E),
                   constant_values=_N).reshape(_RT, 128)

    emb_pad = jnp.zeros((_C, _C), F32).at[:embed.shape[0]].set(embed)

    eye8 = jnp.eye(8, dtype=F32)
    wks = []
    for l in range(1, _CONV_LAYERS):
        w = conv_W[l]
        wks.append(jnp.stack([
            jnp.kron(eye8, w[:_LANE, :_LANE]),
            jnp.kron(eye8, w[:_LANE, _LANE:]),
            jnp.kron(eye8, w[_LANE:, :_LANE]),
            jnp.kron(eye8, w[_LANE:, _LANE:]),
        ]))
    bts = [jnp.stack([jnp.tile(conv_b[l][:_LANE], 8),
                      jnp.tile(conv_b[l][_LANE:], 8)])
           for l in range(_CONV_LAYERS)]

    d0b = dense0_b.reshape(1, _C)
    db = dense_b.reshape(_DENSE_LAYERS, 1, _C)
    fw = jnp.zeros((_C, 8), F32).at[:, :2].set(final_W)
    fb = jnp.full((1, 8), -1e30, F32).at[0, :2].set(final_b)

    degp = _sc_deg(dstr)
    dinvp, hwp = _tc_a(xpk, degp.reshape(_NC, _NPB, 128), emb_pad, conv_W[0])

    for l in range(_CONV_LAYERS):
        acc = _sc_edge(hwp.reshape(_NC, _NP, _LANE), srcr, dstr)
        accp = acc.reshape(_NC, _NPB, 128)
        if l + 1 < _CONV_LAYERS:
            hwp = _tc_b(accp, hwp, dinvp, bts[l], wks[l])
        else:
            out = _tc_seg(accp, hwp, dinvp, batpk, bts[l], dense0_W,
                          d0b, dense_W, db, fw, fb)
    return out


# speedup vs baseline: 40.4317x; 1.0264x over previous
import functools

import jax
import jax.numpy as jnp
from jax import lax
from jax.experimental import pallas as pl
from jax.experimental.pallas import tpu as pltpu
from jax.experimental.pallas import tpu_sc as plsc

F32 = jnp.float32
NEG_INF = float("-inf")

_N = 100000
_E = 1600000
_C = 32
_G = 64
_CONV_LAYERS = 5
_DENSE_LAYERS = 3

_NC = 2
_NS = 16
_LANE = 16

_BLK = 2048
_NP = 100352
_GRID = _NP // _BLK
_NPT = _NP // _NS
_ZCH = 128
_NZ = _NPT // _ZCH

_KCH = 56
_R = 784
_OUTER = _R // _KCH
_RT = _NS * _R
_EP = _RT * 128

_EH = _E // 2
_KCH2 = 56
_R2 = 392
_OUTER2 = _R2 // _KCH2
_RT2 = _NS * _R2
_EP2 = _RT2 * 128


def _fill_rows(ref, nrows, value):
    def body(i, _):
        ref[i, :] = jnp.full((_LANE,), value, F32)
        return 0
    lax.fori_loop(0, nrows, body, 0)


def _sc_mesh():
    return plsc.VectorSubcoreMesh(core_axis_name="c", subcore_axis_name="s")


def _sc_deg(dstr):
    @functools.partial(
        pl.kernel,
        out_type=jax.ShapeDtypeStruct((_NC, _NP, _LANE), F32),
        mesh=_sc_mesh(),
        compiler_params=pltpu.CompilerParams(use_tc_tiling_on_sc=False),
        scratch_types=[
            pltpu.VMEM((_KCH2, 128), jnp.int32),
            pltpu.VMEM((128, _LANE), F32),
            pltpu.VMEM((_ZCH, _LANE), F32),
            pltpu.VMEM_SHARED((_NP, _LANE), F32),
            pltpu.SemaphoreType.DMA,
        ],
    )
    def k(dst_hbm, out_hbm, didx, ones_v, zbuf, acc_sh, dsem):
        c = lax.axis_index("c")
        s = lax.axis_index("s")
        _fill_rows(zbuf, _ZCH, 0.0)
        _fill_rows(ones_v, 128, 1.0)
        base = s * _NPT

        def zero_chunk(m, _):
            pltpu.sync_copy(zbuf, acc_sh.at[pl.ds(base + m * _ZCH, _ZCH)])
            return 0
        lax.fori_loop(0, _NZ, zero_chunk, 0)
        plsc.subcore_barrier()

        rbase = c * (_RT // 2) + s * _R2
        for o in range(_OUTER2):
            pltpu.sync_copy(dst_hbm.at[pl.ds(rbase + o * _KCH2, _KCH2), :],
                            didx)

            def inner(kk, _):
                pltpu.async_copy(ones_v, acc_sh.at[didx.at[kk]], dsem,
                                 add=True)
                return 0
            lax.fori_loop(0, _KCH2, inner, 0)

            def drain(kk, _):
                pltpu.make_async_copy(ones_v, acc_sh.at[didx.at[0]],
                                      dsem).wait()
                return 0
            lax.fori_loop(0, _KCH2, drain, 0)

        plsc.subcore_barrier()

        def copy_out(m, _):
            off = base + m * _ZCH
            pltpu.sync_copy(acc_sh.at[pl.ds(off, _ZCH)], zbuf)
            pltpu.sync_copy(zbuf, out_hbm.at[c, pl.ds(off, _ZCH), :])
            return 0
        lax.fori_loop(0, _NZ, copy_out, 0)

    return k(dstr)


_NBUF = 8
_LA = 4
_NGRP = _KCH // _NBUF


def _sc_edge(hwp, srcr, dstr):
    @functools.partial(
        pl.kernel,
        out_type=jax.ShapeDtypeStruct((_NC, _NP, _LANE), F32),
        mesh=_sc_mesh(),
        compiler_params=pltpu.CompilerParams(use_tc_tiling_on_sc=False),
        scratch_types=[
            pltpu.VMEM((_KCH, 128), jnp.int32),
            pltpu.VMEM((_KCH, 128), jnp.int32),
            pltpu.VMEM((_NBUF, 128, _LANE), F32),
            pltpu.VMEM_SHARED((_NP, _LANE), F32),
            pltpu.SemaphoreType.DMA((_NBUF,)),
            pltpu.SemaphoreType.DMA((_NBUF,)),
        ],
    )
    def k(hwp_hbm, src_hbm, dst_hbm, out_hbm, sidx, didx, rows, acc_sh,
          gsem, ssem):
        c = lax.axis_index("c")
        s = lax.axis_index("s")
        _fill_rows(rows.at[0], _ZCH, 0.0)
        base = s * _NPT

        def zero_chunk(m, _):
            pltpu.sync_copy(rows.at[0],
                            acc_sh.at[pl.ds(base + m * _ZCH, _ZCH)])
            return 0
        lax.fori_loop(0, _NZ, zero_chunk, 0)
        plsc.subcore_barrier()

        def gather(j, b):
            pltpu.async_copy(hwp_hbm.at[c].at[sidx.at[j]], rows.at[b],
                             gsem.at[b])

        def gather_wait(j, b):
            pltpu.make_async_copy(hwp_hbm.at[c].at[sidx.at[j]], rows.at[b],
                                  gsem.at[b]).wait()

        def scat(j, b):
            pltpu.async_copy(rows.at[b], acc_sh.at[didx.at[j]], ssem.at[b],
                             add=True)

        def scat_wait(j, b):
            pltpu.make_async_copy(rows.at[b], acc_sh.at[didx.at[j]],
                                  ssem.at[b]).wait()

        rbase = s * _R
        for o in range(_OUTER):
            if o > 0:
                for u in range(_NBUF):
                    scat_wait(0, u)
            pltpu.sync_copy(src_hbm.at[pl.ds(rbase + o * _KCH, _KCH), :],
                            sidx)
            pltpu.sync_copy(dst_hbm.at[pl.ds(rbase + o * _KCH, _KCH), :],
                            didx)

            for b in range(_LA):
                gather(b, b)

            for u in range(_NBUF):
                gather_wait(u, u)
                scat(u, u)
                bb = (u + _LA) % _NBUF
                if u >= _NBUF - _LA:
                    scat_wait(0, bb)
                gather(u + _LA, bb)

            def group(g, _):
                for u in range(_NBUF):
                    j = g * _NBUF + u
                    gather_wait(j, u)
                    scat(j, u)
                    bb = (u + _LA) % _NBUF
                    scat_wait(0, bb)
                    gather(j + _LA, bb)
                return 0
            lax.fori_loop(1, _NGRP - 1, group, 0)

            for u in range(_NBUF):
                j = (_NGRP - 1) * _NBUF + u
                gather_wait(j, u)
                scat(j, u)
                if u < _NBUF - _LA:
                    bb = (u + _LA) % _NBUF
                    scat_wait(0, bb)
                    gather(j + _LA, bb)

        for u in range(_NBUF):
            scat_wait(0, u)

        plsc.subcore_barrier()

        def copy_out(m, _):
            off = base + m * _ZCH
            pltpu.sync_copy(acc_sh.at[pl.ds(off, _ZCH)], rows.at[0])
            pltpu.sync_copy(rows.at[0], out_hbm.at[c, pl.ds(off, _ZCH), :])
            return 0
        lax.fori_loop(0, _NZ, copy_out, 0)

    return k(hwp, srcr, dstr)


_NPB = _NP // 8
_BLKP = _BLK // 8
_FLAV = 17


def _tc_a_body(x_ref, deg_ref, emb_ref, w_ref, dinv_ref, hwp_ref):
    i = pl.program_id(0)
    dp = deg_ref[0] + deg_ref[1]
    row_iota = lax.broadcasted_iota(jnp.int32, (_BLKP, 128), 0)
    lane_iota = lax.broadcasted_iota(jnp.int32, (_BLKP, 128), 1)
    nid = 8 * (i * _BLKP + row_iota) + lane_iota // _LANE
    dinv = jnp.where(nid < _N, lax.rsqrt(dp + 1.0), 0.0)
    ew = jnp.dot(emb_ref[:], w_ref[:], preferred_element_type=F32)
    ew0 = jnp.concatenate([ew[:, :_LANE]] * 8, axis=1)
    ew1 = jnp.concatenate([ew[:, _LANE:]] * 8, axis=1)
    xq = x_ref[:]
    h0 = jnp.zeros((_BLKP, 128), F32)
    h1 = jnp.zeros((_BLKP, 128), F32)
    for f in range(_FLAV):
        sel = xq == f
        h0 = jnp.where(sel, ew0[f:f + 1, :], h0)
        h1 = jnp.where(sel, ew1[f:f + 1, :], h1)
    dinv_ref[:] = dinv
    hwp_ref[0, :, :] = dinv * h0
    hwp_ref[1, :, :] = dinv * h1


def _tc_a(xpk, degp, emb_pad, w0):
    return pl.pallas_call(
        _tc_a_body,
        grid=(_GRID,),
        in_specs=[
            pl.BlockSpec((_BLKP, 128), lambda i: (i, 0)),
            pl.BlockSpec((_NC, _BLKP, 128), lambda i: (0, i, 0)),
            pl.BlockSpec((_C, _C), lambda i: (0, 0)),
            pl.BlockSpec((_C, _C), lambda i: (0, 0)),
        ],
        out_specs=[
            pl.BlockSpec((_BLKP, 128), lambda i: (i, 0)),
            pl.BlockSpec((_NC, _BLKP, 128), lambda i: (0, i, 0)),
        ],
        out_shape=[
            jax.ShapeDtypeStruct((_NPB, 128), F32),
            jax.ShapeDtypeStruct((_NC, _NPB, 128), F32),
        ],
    )(xpk, degp, emb_pad, w0)


def _layer_h(acc_ref, hwp_ref, dinv_ref, b_ref):
    dinv = dinv_ref[:]
    h0 = jnp.maximum(dinv * (acc_ref[0] + hwp_ref[0]) + b_ref[0:1, :], 0.0)
    h1 = jnp.maximum(dinv * (acc_ref[1] + hwp_ref[1]) + b_ref[1:2, :], 0.0)
    return dinv, h0, h1


def _tc_b_body(acc_ref, hwp_ref, dinv_ref, b_ref, wk_ref, out_ref):
    dinv, h0, h1 = _layer_h(acc_ref, hwp_ref, dinv_ref, b_ref)
    hw0 = (jnp.dot(h0, wk_ref[0], preferred_element_type=F32)
           + jnp.dot(h1, wk_ref[2], preferred_element_type=F32))
    hw1 = (jnp.dot(h0, wk_ref[1], preferred_element_type=F32)
           + jnp.dot(h1, wk_ref[3], preferred_element_type=F32))
    out_ref[0, :, :] = dinv * hw0
    out_ref[1, :, :] = dinv * hw1


_BLKPB = 448
_GRIDB = _NPB // _BLKPB


def _tc_b(acc, hwp, dinvp, bt, wk):
    return pl.pallas_call(
        _tc_b_body,
        grid=(_GRIDB,),
        in_specs=[
            pl.BlockSpec((_NC, _BLKPB, 128), lambda i: (0, i, 0)),
            pl.BlockSpec((_NC, _BLKPB, 128), lambda i: (0, i, 0)),
            pl.BlockSpec((_BLKPB, 128), lambda i: (i, 0)),
            pl.BlockSpec((2, 128), lambda i: (0, 0)),
            pl.BlockSpec((4, 128, 128), lambda i: (0, 0, 0)),
        ],
        out_specs=pl.BlockSpec((_NC, _BLKPB, 128), lambda i: (0, i, 0)),
        out_shape=jax.ShapeDtypeStruct((_NC, _NPB, 128), F32),
    )(acc, hwp, dinvp, bt, wk)


def _tc_seg_body(acc_ref, hwp_ref, dinv_ref, bat_ref, b_ref, d0w_ref,
                 d0b_ref, dw_ref, db_ref, fw_ref, fb_ref, out_ref, smax_ref):
    i = pl.program_id(0)

    @pl.when(i == 0)
    def _():
        smax_ref[:] = jnp.full((_G + 8, _C), NEG_INF, F32)

    _, h0, h1 = _layer_h(acc_ref, hwp_ref, dinv_ref, b_ref)

    bi = bat_ref[:]
    g_first = bat_ref[0, 0]
    g_last = bat_ref[_BLKP - 1, 127]

    def upd(g, _):
        m0 = jnp.max(jnp.where(bi == g, h0, NEG_INF), axis=0, keepdims=True)
        m1 = jnp.max(jnp.where(bi == g, h1, NEG_INF), axis=0, keepdims=True)
        r0 = m0[:, 0:_LANE]
        r1 = m1[:, 0:_LANE]
        for k in range(1, 8):
            r0 = jnp.maximum(r0, m0[:, k * _LANE:(k + 1) * _LANE])
            r1 = jnp.maximum(r1, m1[:, k * _LANE:(k + 1) * _LANE])
        m = jnp.concatenate([r0, r1], axis=1)
        cur = smax_ref[pl.ds(g, 1), :]
        smax_ref[pl.ds(g, 1), :] = jnp.maximum(cur, m)
        return 0
    lax.fori_loop(g_first, g_last + 1, upd, 0)

    @pl.when(i == _GRID - 1)
    def _():
        g = smax_ref[0:_G, :]
        g = jnp.maximum(
            jnp.dot(g, d0w_ref[:], preferred_element_type=F32) + d0b_ref[:],
            0.0)
        for j in range(_DENSE_LAYERS):
            g = jnp.maximum(
                jnp.dot(g, dw_ref[j], preferred_element_type=F32)
                + db_ref[j], 0.0)
        logits = jnp.dot(g, fw_ref[:], preferred_element_type=F32) + fb_ref[:]
        m = jnp.max(logits, axis=1, keepdims=True)
        z = logits - m
        lse = jnp.log(jnp.sum(jnp.exp(z), axis=1, keepdims=True))
        out_ref[:] = (z - lse)[:, 0:2]


def _tc_seg(acc, hwp, dinvp, batpk, bt, d0w, d0b, dw, db, fw, fb):
    return pl.pallas_call(
        _tc_seg_body,
        grid=(_GRID,),
        in_specs=[
            pl.BlockSpec((_NC, _BLKP, 128), lambda i: (0, i, 0)),
            pl.BlockSpec((_NC, _BLKP, 128), lambda i: (0, i, 0)),
            pl.BlockSpec((_BLKP, 128), lambda i: (i, 0)),
            pl.BlockSpec((_BLKP, 128), lambda i: (i, 0)),
            pl.BlockSpec((2, 128), lambda i: (0, 0)),
            pl.BlockSpec((_C, _C), lambda i: (0, 0)),
            pl.BlockSpec((1, _C), lambda i: (0, 0)),
            pl.BlockSpec((_DENSE_LAYERS, _C, _C), lambda i: (0, 0, 0)),
            pl.BlockSpec((_DENSE_LAYERS, 1, _C), lambda i: (0, 0, 0)),
            pl.BlockSpec((_C, 8), lambda i: (0, 0)),
            pl.BlockSpec((1, 8), lambda i: (0, 0)),
        ],
        out_specs=pl.BlockSpec((_G, 2), lambda i: (0, 0)),
        out_shape=jax.ShapeDtypeStruct((_G, 2), F32),
        scratch_shapes=[pltpu.VMEM((_G + 8, _C), F32)],
    )(acc, hwp, dinvp, batpk, bt, d0w, d0b, dw, db, fw, fb)


def _pack_scalar(v, pad_value):
    vp = jnp.pad(v, (0, _NP - _N), constant_values=pad_value)
    return jnp.repeat(vp, _LANE).reshape(_NPB, 128)


def kernel(x, edge_index, batch, embed, conv_W, conv_b, dense0_W, dense0_b,
           dense_W, dense_b, final_W, final_b):
    x32 = x.astype(jnp.int32)
    src = edge_index[0].astype(jnp.int32)
    dst = edge_index[1].astype(jnp.int32)
    bat = batch.astype(jnp.int32)

    xpk = _pack_scalar(x32, 0)
    batpk = _pack_scalar(bat, _G)

    srcr = jnp.pad(src, (0, _EP - _E), constant_values=_N).reshape(_RT, 128)
    dstr = jnp.pad(dst, (0, _EP - _E),
                   constant_values=_N).reshape(_RT, 128)

    emb_pad = jnp.zeros((_C, _C), F32).at[:embed.shape[0]].set(embed)

    eye8 = jnp.eye(8, dtype=F32)
    wks = []
    for l in range(1, _CONV_LAYERS):
        w = conv_W[l]
        wks.append(jnp.stack([
            jnp.kron(eye8, w[:_LANE, :_LANE]),
            jnp.kron(eye8, w[:_LANE, _LANE:]),
            jnp.kron(eye8, w[_LANE:, :_LANE]),
            jnp.kron(eye8, w[_LANE:, _LANE:]),
        ]))
    bts = [jnp.stack([jnp.tile(conv_b[l][:_LANE], 8),
                      jnp.tile(conv_b[l][_LANE:], 8)])
           for l in range(_CONV_LAYERS)]

    d0b = dense0_b.reshape(1, _C)
    db = dense_b.reshape(_DENSE_LAYERS, 1, _C)
    fw = jnp.zeros((_C, 8), F32).at[:, :2].set(final_W)
    fb = jnp.full((1, 8), -1e30, F32).at[0, :2].set(final_b)

    degp = _sc_deg(dstr)
    dinvp, hwp = _tc_a(xpk, degp.reshape(_NC, _NPB, 128), emb_pad, conv_W[0])

    for l in range(_CONV_LAYERS):
        acc = _sc_edge(hwp.reshape(_NC, _NP, _LANE), srcr, dstr)
        accp = acc.reshape(_NC, _NPB, 128)
        if l + 1 < _CONV_LAYERS:
            hwp = _tc_b(accp, hwp, dinvp, bts[l], wks[l])
        else:
            out = _tc_seg(accp, hwp, dinvp, batpk, bts[l], dense0_W,
                          d0b, dense_W, db, fw, fb)
    return out


# speedup vs baseline: 40.9627x; 1.0131x over previous
import functools

import jax
import jax.numpy as jnp
from jax import lax
from jax.experimental import pallas as pl
from jax.experimental.pallas import tpu as pltpu
from jax.experimental.pallas import tpu_sc as plsc

F32 = jnp.float32
NEG_INF = float("-inf")

_N = 100000
_E = 1600000
_C = 32
_G = 64
_CONV_LAYERS = 5
_DENSE_LAYERS = 3

_NC = 2
_NS = 16
_LANE = 16

_BLK = 2048
_NP = 100352
_GRID = _NP // _BLK
_NPT = _NP // _NS
_ZCH = 128
_NZ = _NPT // _ZCH

_KCH = 56
_R = 784
_OUTER = _R // _KCH
_RT = _NS * _R
_EP = _RT * 128

_EH = _E // 2
_KCH2 = 56
_R2 = 392
_OUTER2 = _R2 // _KCH2
_RT2 = _NS * _R2
_EP2 = _RT2 * 128


def _fill_rows(ref, nrows, value):
    def body(i, _):
        ref[i, :] = jnp.full((_LANE,), value, F32)
        return 0
    lax.fori_loop(0, nrows, body, 0)


def _sc_mesh():
    return plsc.VectorSubcoreMesh(core_axis_name="c", subcore_axis_name="s")


def _sc_deg(dstr):
    @functools.partial(
        pl.kernel,
        out_type=jax.ShapeDtypeStruct((_NC, _NP, _LANE), F32),
        mesh=_sc_mesh(),
        compiler_params=pltpu.CompilerParams(use_tc_tiling_on_sc=False),
        scratch_types=[
            pltpu.VMEM((_KCH2, 128), jnp.int32),
            pltpu.VMEM((128, _LANE), F32),
            pltpu.VMEM((_ZCH, _LANE), F32),
            pltpu.VMEM_SHARED((_NP, _LANE), F32),
            pltpu.SemaphoreType.DMA,
        ],
    )
    def k(dst_hbm, out_hbm, didx, ones_v, zbuf, acc_sh, dsem):
        c = lax.axis_index("c")
        s = lax.axis_index("s")
        _fill_rows(zbuf, _ZCH, 0.0)
        _fill_rows(ones_v, 128, 1.0)
        base = s * _NPT

        def zero_chunk(m, _):
            pltpu.sync_copy(zbuf, acc_sh.at[pl.ds(base + m * _ZCH, _ZCH)])
            return 0
        lax.fori_loop(0, _NZ, zero_chunk, 0)
        plsc.subcore_barrier()

        rbase = c * (_RT // 2) + s * _R2
        for o in range(_OUTER2):
            pltpu.sync_copy(dst_hbm.at[pl.ds(rbase + o * _KCH2, _KCH2), :],
                            didx)

            def inner(kk, _):
                pltpu.async_copy(ones_v, acc_sh.at[didx.at[kk]], dsem,
                                 add=True)
                return 0
            lax.fori_loop(0, _KCH2, inner, 0)

            def drain(kk, _):
                pltpu.make_async_copy(ones_v, acc_sh.at[didx.at[0]],
                                      dsem).wait()
                return 0
            lax.fori_loop(0, _KCH2, drain, 0)

        plsc.subcore_barrier()

        def copy_out(m, _):
            off = base + m * _ZCH
            pltpu.sync_copy(acc_sh.at[pl.ds(off, _ZCH)], zbuf)
            pltpu.sync_copy(zbuf, out_hbm.at[c, pl.ds(off, _ZCH), :])
            return 0
        lax.fori_loop(0, _NZ, copy_out, 0)

    return k(dstr)


_NBUF = 8
_LA = 4
_NGRP = _KCH // _NBUF


def _sc_edge(hwp, srcr, dstr):
    @functools.partial(
        pl.kernel,
        out_type=jax.ShapeDtypeStruct((_NC, _NP, _LANE), F32),
        mesh=_sc_mesh(),
        compiler_params=pltpu.CompilerParams(use_tc_tiling_on_sc=False),
        scratch_types=[
            pltpu.VMEM((_KCH, 128), jnp.int32),
            pltpu.VMEM((_KCH, 128), jnp.int32),
            pltpu.VMEM((_NBUF, 128, _LANE), F32),
            pltpu.VMEM_SHARED((_NP, _LANE), F32),
            pltpu.SemaphoreType.DMA((_NBUF,)),
            pltpu.SemaphoreType.DMA((_NBUF,)),
        ],
    )
    def k(hwp_hbm, src_hbm, dst_hbm, out_hbm, sidx, didx, rows, acc_sh,
          gsem, ssem):
        c = lax.axis_index("c")
        s = lax.axis_index("s")
        _fill_rows(rows.at[0], _ZCH, 0.0)
        base = s * _NPT

        def zero_chunk(m, _):
            pltpu.sync_copy(rows.at[0],
                            acc_sh.at[pl.ds(base + m * _ZCH, _ZCH)])
            return 0
        lax.fori_loop(0, _NZ, zero_chunk, 0)
        plsc.subcore_barrier()

        def gather(j, b):
            pltpu.async_copy(hwp_hbm.at[c].at[sidx.at[j]], rows.at[b],
                             gsem.at[b])

        def gather_wait(j, b):
            pltpu.make_async_copy(hwp_hbm.at[c].at[sidx.at[j]], rows.at[b],
                                  gsem.at[b]).wait()

        def scat(j, b):
            pltpu.async_copy(rows.at[b], acc_sh.at[didx.at[j]], ssem.at[b],
                             add=True)

        def scat_wait(j, b):
            pltpu.make_async_copy(rows.at[b], acc_sh.at[didx.at[j]],
                                  ssem.at[b]).wait()

        rbase = s * _R
        for o in range(_OUTER):
            if o > 0:
                for u in range(_NBUF):
                    scat_wait(0, u)
            pltpu.sync_copy(src_hbm.at[pl.ds(rbase + o * _KCH, _KCH), :],
                            sidx)
            pltpu.sync_copy(dst_hbm.at[pl.ds(rbase + o * _KCH, _KCH), :],
                            didx)

            for b in range(_LA):
                gather(b, b)

            for u in range(_NBUF):
                gather_wait(u, u)
                scat(u, u)
                bb = (u + _LA) % _NBUF
                if u >= _NBUF - _LA:
                    scat_wait(0, bb)
                gather(u + _LA, bb)

            def group(g, _):
                for u in range(_NBUF):
                    j = g * _NBUF + u
                    gather_wait(j, u)
                    scat(j, u)
                    bb = (u + _LA) % _NBUF
                    scat_wait(0, bb)
                    gather(j + _LA, bb)
                return 0
            lax.fori_loop(1, _NGRP - 1, group, 0)

            for u in range(_NBUF):
                j = (_NGRP - 1) * _NBUF + u
                gather_wait(j, u)
                scat(j, u)
                if u < _NBUF - _LA:
                    bb = (u + _LA) % _NBUF
                    scat_wait(0, bb)
                    gather(j + _LA, bb)

        for u in range(_NBUF):
            scat_wait(0, u)

        plsc.subcore_barrier()

        def copy_out(m, _):
            off = base + m * _ZCH
            pltpu.sync_copy(acc_sh.at[pl.ds(off, _ZCH)], rows.at[0])
            pltpu.sync_copy(rows.at[0], out_hbm.at[c, pl.ds(off, _ZCH), :])
            return 0
        lax.fori_loop(0, _NZ, copy_out, 0)

    return k(hwp, srcr, dstr)


_NPB = _NP // 8
_BLKP = 448
_GRIDP = _NPB // _BLKP
_FLAV = 17


def _tc_a_body(x_ref, deg_ref, emb_ref, w_ref, dinv_ref, hwp_ref):
    i = pl.program_id(0)
    dp = deg_ref[0] + deg_ref[1]
    row_iota = lax.broadcasted_iota(jnp.int32, (_BLKP, 128), 0)
    lane_iota = lax.broadcasted_iota(jnp.int32, (_BLKP, 128), 1)
    nid = 8 * (i * _BLKP + row_iota) + lane_iota // _LANE
    dinv = jnp.where(nid < _N, lax.rsqrt(dp + 1.0), 0.0)
    ew = jnp.dot(emb_ref[:], w_ref[:], preferred_element_type=F32)
    ew0 = jnp.concatenate([ew[:, :_LANE]] * 8, axis=1)
    ew1 = jnp.concatenate([ew[:, _LANE:]] * 8, axis=1)
    xq = x_ref[:]
    h0 = jnp.zeros((_BLKP, 128), F32)
    h1 = jnp.zeros((_BLKP, 128), F32)
    for f in range(_FLAV):
        sel = xq == f
        h0 = jnp.where(sel, ew0[f:f + 1, :], h0)
        h1 = jnp.where(sel, ew1[f:f + 1, :], h1)
    dinv_ref[:] = dinv
    hwp_ref[0, :, :] = dinv * h0
    hwp_ref[1, :, :] = dinv * h1


def _tc_a(xpk, degp, emb_pad, w0):
    return pl.pallas_call(
        _tc_a_body,
        grid=(_GRIDP,),
        in_specs=[
            pl.BlockSpec((_BLKP, 128), lambda i: (i, 0)),
            pl.BlockSpec((_NC, _BLKP, 128), lambda i: (0, i, 0)),
            pl.BlockSpec((_C, _C), lambda i: (0, 0)),
            pl.BlockSpec((_C, _C), lambda i: (0, 0)),
        ],
        out_specs=[
            pl.BlockSpec((_BLKP, 128), lambda i: (i, 0)),
            pl.BlockSpec((_NC, _BLKP, 128), lambda i: (0, i, 0)),
        ],
        out_shape=[
            jax.ShapeDtypeStruct((_NPB, 128), F32),
            jax.ShapeDtypeStruct((_NC, _NPB, 128), F32),
        ],
    )(xpk, degp, emb_pad, w0)


def _layer_h(acc_ref, hwp_ref, dinv_ref, b_ref):
    dinv = dinv_ref[:]
    h0 = jnp.maximum(dinv * (acc_ref[0] + hwp_ref[0]) + b_ref[0:1, :], 0.0)
    h1 = jnp.maximum(dinv * (acc_ref[1] + hwp_ref[1]) + b_ref[1:2, :], 0.0)
    return dinv, h0, h1


def _tc_b_body(acc_ref, hwp_ref, dinv_ref, b_ref, wk_ref, out_ref):
    dinv, h0, h1 = _layer_h(acc_ref, hwp_ref, dinv_ref, b_ref)
    hw0 = (jnp.dot(h0, wk_ref[0], preferred_element_type=F32)
           + jnp.dot(h1, wk_ref[2], preferred_element_type=F32))
    hw1 = (jnp.dot(h0, wk_ref[1], preferred_element_type=F32)
           + jnp.dot(h1, wk_ref[3], preferred_element_type=F32))
    out_ref[0, :, :] = dinv * hw0
    out_ref[1, :, :] = dinv * hw1


def _tc_b(acc, hwp, dinvp, bt, wk):
    return pl.pallas_call(
        _tc_b_body,
        grid=(_GRIDP,),
        in_specs=[
            pl.BlockSpec((_NC, _BLKP, 128), lambda i: (0, i, 0)),
            pl.BlockSpec((_NC, _BLKP, 128), lambda i: (0, i, 0)),
            pl.BlockSpec((_BLKP, 128), lambda i: (i, 0)),
            pl.BlockSpec((2, 128), lambda i: (0, 0)),
            pl.BlockSpec((4, 128, 128), lambda i: (0, 0, 0)),
        ],
        out_specs=pl.BlockSpec((_NC, _BLKP, 128), lambda i: (0, i, 0)),
        out_shape=jax.ShapeDtypeStruct((_NC, _NPB, 128), F32),
    )(acc, hwp, dinvp, bt, wk)


def _tc_seg_body(acc_ref, hwp_ref, dinv_ref, bat_ref, b_ref, d0w_ref,
                 d0b_ref, dw_ref, db_ref, fw_ref, fb_ref, out_ref, smax_ref):
    i = pl.program_id(0)

    @pl.when(i == 0)
    def _():
        smax_ref[:] = jnp.full((_G + 8, _C), NEG_INF, F32)

    _, h0, h1 = _layer_h(acc_ref, hwp_ref, dinv_ref, b_ref)

    bi = bat_ref[:]
    g_first = bat_ref[0, 0]
    g_last = bat_ref[_BLKP - 1, 127]

    def upd(g, _):
        m0 = jnp.max(jnp.where(bi == g, h0, NEG_INF), axis=0, keepdims=True)
        m1 = jnp.max(jnp.where(bi == g, h1, NEG_INF), axis=0, keepdims=True)
        r0 = m0[:, 0:_LANE]
        r1 = m1[:, 0:_LANE]
        for k in range(1, 8):
            r0 = jnp.maximum(r0, m0[:, k * _LANE:(k + 1) * _LANE])
            r1 = jnp.maximum(r1, m1[:, k * _LANE:(k + 1) * _LANE])
        m = jnp.concatenate([r0, r1], axis=1)
        cur = smax_ref[pl.ds(g, 1), :]
        smax_ref[pl.ds(g, 1), :] = jnp.maximum(cur, m)
        return 0
    lax.fori_loop(g_first, g_last + 1, upd, 0)

    @pl.when(i == _GRIDP - 1)
    def _():
        g = smax_ref[0:_G, :]
        g = jnp.maximum(
            jnp.dot(g, d0w_ref[:], preferred_element_type=F32) + d0b_ref[:],
            0.0)
        for j in range(_DENSE_LAYERS):
            g = jnp.maximum(
                jnp.dot(g, dw_ref[j], preferred_element_type=F32)
                + db_ref[j], 0.0)
        logits = jnp.dot(g, fw_ref[:], preferred_element_type=F32) + fb_ref[:]
        m = jnp.max(logits, axis=1, keepdims=True)
        z = logits - m
        lse = jnp.log(jnp.sum(jnp.exp(z), axis=1, keepdims=True))
        out_ref[:] = (z - lse)[:, 0:2]


def _tc_seg(acc, hwp, dinvp, batpk, bt, d0w, d0b, dw, db, fw, fb):
    return pl.pallas_call(
        _tc_seg_body,
        grid=(_GRIDP,),
        in_specs=[
            pl.BlockSpec((_NC, _BLKP, 128), lambda i: (0, i, 0)),
            pl.BlockSpec((_NC, _BLKP, 128), lambda i: (0, i, 0)),
            pl.BlockSpec((_BLKP, 128), lambda i: (i, 0)),
            pl.BlockSpec((_BLKP, 128), lambda i: (i, 0)),
            pl.BlockSpec((2, 128), lambda i: (0, 0)),
            pl.BlockSpec((_C, _C), lambda i: (0, 0)),
            pl.BlockSpec((1, _C), lambda i: (0, 0)),
            pl.BlockSpec((_DENSE_LAYERS, _C, _C), lambda i: (0, 0, 0)),
            pl.BlockSpec((_DENSE_LAYERS, 1, _C), lambda i: (0, 0, 0)),
            pl.BlockSpec((_C, 8), lambda i: (0, 0)),
            pl.BlockSpec((1, 8), lambda i: (0, 0)),
        ],
        out_specs=pl.BlockSpec((_G, 2), lambda i: (0, 0)),
        out_shape=jax.ShapeDtypeStruct((_G, 2), F32),
        scratch_shapes=[pltpu.VMEM((_G + 8, _C), F32)],
    )(acc, hwp, dinvp, batpk, bt, d0w, d0b, dw, db, fw, fb)


def _pack_scalar(v, pad_value):
    vp = jnp.pad(v, (0, _NP - _N), constant_values=pad_value)
    return jnp.repeat(vp, _LANE).reshape(_NPB, 128)


def kernel(x, edge_index, batch, embed, conv_W, conv_b, dense0_W, dense0_b,
           dense_W, dense_b, final_W, final_b):
    x32 = x.astype(jnp.int32)
    src = edge_index[0].astype(jnp.int32)
    dst = edge_index[1].astype(jnp.int32)
    bat = batch.astype(jnp.int32)

    xpk = _pack_scalar(x32, 0)
    batpk = _pack_scalar(bat, _G)

    srcr = jnp.pad(src, (0, _EP - _E), constant_values=_N).reshape(_RT, 128)
    dstr = jnp.pad(dst, (0, _EP - _E),
                   constant_values=_N).reshape(_RT, 128)

    emb_pad = jnp.zeros((_C, _C), F32).at[:embed.shape[0]].set(embed)

    eye8 = jnp.eye(8, dtype=F32)
    wks = []
    for l in range(1, _CONV_LAYERS):
        w = conv_W[l]
        wks.append(jnp.stack([
            jnp.kron(eye8, w[:_LANE, :_LANE]),
            jnp.kron(eye8, w[:_LANE, _LANE:]),
            jnp.kron(eye8, w[_LANE:, :_LANE]),
            jnp.kron(eye8, w[_LANE:, _LANE:]),
        ]))
    bts = [jnp.stack([jnp.tile(conv_b[l][:_LANE], 8),
                      jnp.tile(conv_b[l][_LANE:], 8)])
           for l in range(_CONV_LAYERS)]

    d0b = dense0_b.reshape(1, _C)
    db = dense_b.reshape(_DENSE_LAYERS, 1, _C)
    fw = jnp.zeros((_C, 8), F32).at[:, :2].set(final_W)
    fb = jnp.full((1, 8), -1e30, F32).at[0, :2].set(final_b)

    degp = _sc_deg(dstr)
    dinvp, hwp = _tc_a(xpk, degp.reshape(_NC, _NPB, 128), emb_pad, conv_W[0])

    for l in range(_CONV_LAYERS):
        acc = _sc_edge(hwp.reshape(_NC, _NP, _LANE), srcr, dstr)
        accp = acc.reshape(_NC, _NPB, 128)
        if l + 1 < _CONV_LAYERS:
            hwp = _tc_b(accp, hwp, dinvp, bts[l], wks[l])
        else:
            out = _tc_seg(accp, hwp, dinvp, batpk, bts[l], dense0_W,
                          d0b, dense_W, db, fw, fb)
    return out


# speedup vs baseline: 46.6844x; 1.1397x over previous
import functools

import jax
import jax.numpy as jnp
from jax import lax
from jax.experimental import pallas as pl
from jax.experimental.pallas import tpu as pltpu
from jax.experimental.pallas import tpu_sc as plsc

F32 = jnp.float32
NEG_INF = float("-inf")

_N = 100000
_E = 1600000
_C = 32
_G = 64
_CONV_LAYERS = 5
_DENSE_LAYERS = 3

_NC = 2
_NS = 16
_LANE = 16

_BLK = 2048
_NP = 100352
_GRID = _NP // _BLK
_NPT = _NP // _NS
_ZCH = 128
_NZ = _NPT // _ZCH

_KCH = 56
_R = 784
_OUTER = _R // _KCH
_RT = _NS * _R
_EP = _RT * 128

_EH = _E // 2
_KCH2 = 56
_R2 = 392
_OUTER2 = _R2 // _KCH2
_RT2 = _NS * _R2
_EP2 = _RT2 * 128


def _fill_rows(ref, nrows, value):
    def body(i, _):
        ref[i, :] = jnp.full((_LANE,), value, F32)
        return 0
    lax.fori_loop(0, nrows, body, 0)


def _sc_mesh():
    return plsc.VectorSubcoreMesh(core_axis_name="c", subcore_axis_name="s")


def _sc_deg(dstr):
    @functools.partial(
        pl.kernel,
        out_type=jax.ShapeDtypeStruct((_NC, _NP, _LANE), F32),
        mesh=_sc_mesh(),
        compiler_params=pltpu.CompilerParams(use_tc_tiling_on_sc=False),
        scratch_types=[
            pltpu.VMEM((_KCH2, 128), jnp.int32),
            pltpu.VMEM((128, _LANE), F32),
            pltpu.VMEM((_ZCH, _LANE), F32),
            pltpu.VMEM_SHARED((_NP, _LANE), F32),
            pltpu.SemaphoreType.DMA,
        ],
    )
    def k(dst_hbm, out_hbm, didx, ones_v, zbuf, acc_sh, dsem):
        c = lax.axis_index("c")
        s = lax.axis_index("s")
        _fill_rows(zbuf, _ZCH, 0.0)
        _fill_rows(ones_v, 128, 1.0)
        base = s * _NPT

        def zero_chunk(m, _):
            pltpu.sync_copy(zbuf, acc_sh.at[pl.ds(base + m * _ZCH, _ZCH)])
            return 0
        lax.fori_loop(0, _NZ, zero_chunk, 0)
        plsc.subcore_barrier()

        rbase = c * (_RT // 2) + s * _R2
        for o in range(_OUTER2):
            pltpu.sync_copy(dst_hbm.at[pl.ds(rbase + o * _KCH2, _KCH2), :],
                            didx)

            def inner(kk, _):
                pltpu.async_copy(ones_v, acc_sh.at[didx.at[kk]], dsem,
                                 add=True)
                return 0
            lax.fori_loop(0, _KCH2, inner, 0)

            def drain(kk, _):
                pltpu.make_async_copy(ones_v, acc_sh.at[didx.at[0]],
                                      dsem).wait()
                return 0
            lax.fori_loop(0, _KCH2, drain, 0)

        plsc.subcore_barrier()

        def copy_out(m, _):
            off = base + m * _ZCH
            pltpu.sync_copy(acc_sh.at[pl.ds(off, _ZCH)], zbuf)
            pltpu.sync_copy(zbuf, out_hbm.at[c, pl.ds(off, _ZCH), :])
            return 0
        lax.fori_loop(0, _NZ, copy_out, 0)

    return k(dstr)


_NBUF = 8
_LA = 6
_NGRP = _KCH // _NBUF


def _sc_edge(hwp, srcr, dstr):
    @functools.partial(
        pl.kernel,
        out_type=jax.ShapeDtypeStruct((_NC, _NP, _LANE), F32),
        mesh=_sc_mesh(),
        compiler_params=pltpu.CompilerParams(use_tc_tiling_on_sc=False),
        scratch_types=[
            pltpu.VMEM((_KCH, 128), jnp.int32),
            pltpu.VMEM((_KCH, 128), jnp.int32),
            pltpu.VMEM((_NBUF, 128, _LANE), F32),
            pltpu.VMEM_SHARED((_NP, _LANE), F32),
            pltpu.SemaphoreType.DMA((_NBUF,)),
            pltpu.SemaphoreType.DMA((_NBUF,)),
        ],
    )
    def k(hwp_hbm, src_hbm, dst_hbm, out_hbm, sidx, didx, rows, acc_sh,
          gsem, ssem):
        c = lax.axis_index("c")
        s = lax.axis_index("s")
        _fill_rows(rows.at[0], _ZCH, 0.0)
        base = s * _NPT

        def zero_chunk(m, _):
            pltpu.sync_copy(rows.at[0],
                            acc_sh.at[pl.ds(base + m * _ZCH, _ZCH)])
            return 0
        lax.fori_loop(0, _NZ, zero_chunk, 0)
        plsc.subcore_barrier()

        def gather(j, b):
            pltpu.async_copy(hwp_hbm.at[c].at[sidx.at[j]], rows.at[b],
                             gsem.at[b])

        def gather_wait(j, b):
            pltpu.make_async_copy(hwp_hbm.at[c].at[sidx.at[j]], rows.at[b],
                                  gsem.at[b]).wait()

        def scat(j, b):
            pltpu.async_copy(rows.at[b], acc_sh.at[didx.at[j]], ssem.at[b],
                             add=True)

        def scat_wait(j, b):
            pltpu.make_async_copy(rows.at[b], acc_sh.at[didx.at[j]],
                                  ssem.at[b]).wait()

        rbase = s * _R
        for o in range(_OUTER):
            if o > 0:
                for u in range(_NBUF):
                    scat_wait(0, u)
            pltpu.sync_copy(src_hbm.at[pl.ds(rbase + o * _KCH, _KCH), :],
                            sidx)
            pltpu.sync_copy(dst_hbm.at[pl.ds(rbase + o * _KCH, _KCH), :],
                            didx)

            for b in range(_LA):
                gather(b, b)

            for u in range(_NBUF):
                gather_wait(u, u)
                scat(u, u)
                bb = (u + _LA) % _NBUF
                if u >= _NBUF - _LA:
                    scat_wait(0, bb)
                gather(u + _LA, bb)

            def group(g, _):
                for u in range(_NBUF):
                    j = g * _NBUF + u
                    gather_wait(j, u)
                    scat(j, u)
                    bb = (u + _LA) % _NBUF
                    scat_wait(0, bb)
                    gather(j + _LA, bb)
                return 0
            lax.fori_loop(1, _NGRP - 1, group, 0)

            for u in range(_NBUF):
                j = (_NGRP - 1) * _NBUF + u
                gather_wait(j, u)
                scat(j, u)
                if u < _NBUF - _LA:
                    bb = (u + _LA) % _NBUF
                    scat_wait(0, bb)
                    gather(j + _LA, bb)

        for u in range(_NBUF):
            scat_wait(0, u)

        plsc.subcore_barrier()

        def copy_out(m, _):
            off = base + m * _ZCH
            pltpu.sync_copy(acc_sh.at[pl.ds(off, _ZCH)], rows.at[0])
            pltpu.sync_copy(rows.at[0], out_hbm.at[c, pl.ds(off, _ZCH), :])
            return 0
        lax.fori_loop(0, _NZ, copy_out, 0)

    return k(hwp, srcr, dstr)


_NPB = _NP // 8
_BLKP = 448
_GRIDP = _NPB // _BLKP
_FLAV = 17


def _tc_a_body(x_ref, deg_ref, emb_ref, w_ref, dinv_ref, hwp_ref):
    i = pl.program_id(0)
    dp = deg_ref[0] + deg_ref[1]
    row_iota = lax.broadcasted_iota(jnp.int32, (_BLKP, 128), 0)
    lane_iota = lax.broadcasted_iota(jnp.int32, (_BLKP, 128), 1)
    nid = 8 * (i * _BLKP + row_iota) + lane_iota // _LANE
    dinv = jnp.where(nid < _N, lax.rsqrt(dp + 1.0), 0.0)
    ew = jnp.dot(emb_ref[:], w_ref[:], preferred_element_type=F32)
    ew0 = jnp.concatenate([ew[:, :_LANE]] * 8, axis=1)
    ew1 = jnp.concatenate([ew[:, _LANE:]] * 8, axis=1)
    xq = x_ref[:]
    h0 = jnp.zeros((_BLKP, 128), F32)
    h1 = jnp.zeros((_BLKP, 128), F32)
    for f in range(_FLAV):
        sel = xq == f
        h0 = jnp.where(sel, ew0[f:f + 1, :], h0)
        h1 = jnp.where(sel, ew1[f:f + 1, :], h1)
    dinv_ref[:] = dinv
    hwp_ref[0, :, :] = dinv * h0
    hwp_ref[1, :, :] = dinv * h1


def _tc_a(xpk, degp, emb_pad, w0):
    return pl.pallas_call(
        _tc_a_body,
        grid=(_GRIDP,),
        in_specs=[
            pl.BlockSpec((_BLKP, 128), lambda i: (i, 0)),
            pl.BlockSpec((_NC, _BLKP, 128), lambda i: (0, i, 0)),
            pl.BlockSpec((_C, _C), lambda i: (0, 0)),
            pl.BlockSpec((_C, _C), lambda i: (0, 0)),
        ],
        out_specs=[
            pl.BlockSpec((_BLKP, 128), lambda i: (i, 0)),
            pl.BlockSpec((_NC, _BLKP, 128), lambda i: (0, i, 0)),
        ],
        out_shape=[
            jax.ShapeDtypeStruct((_NPB, 128), F32),
            jax.ShapeDtypeStruct((_NC, _NPB, 128), F32),
        ],
    )(xpk, degp, emb_pad, w0)


def _layer_h(acc_ref, hwp_ref, dinv_ref, b_ref):
    dinv = dinv_ref[:]
    h0 = jnp.maximum(dinv * (acc_ref[0] + hwp_ref[0]) + b_ref[0:1, :], 0.0)
    h1 = jnp.maximum(dinv * (acc_ref[1] + hwp_ref[1]) + b_ref[1:2, :], 0.0)
    return dinv, h0, h1


def _tc_b_body(acc_ref, hwp_ref, dinv_ref, b_ref, wk_ref, out_ref):
    dinv, h0, h1 = _layer_h(acc_ref, hwp_ref, dinv_ref, b_ref)
    hw0 = (jnp.dot(h0, wk_ref[0], preferred_element_type=F32)
           + jnp.dot(h1, wk_ref[2], preferred_element_type=F32))
    hw1 = (jnp.dot(h0, wk_ref[1], preferred_element_type=F32)
           + jnp.dot(h1, wk_ref[3], preferred_element_type=F32))
    out_ref[0, :, :] = dinv * hw0
    out_ref[1, :, :] = dinv * hw1


def _tc_b(acc, hwp, dinvp, bt, wk):
    return pl.pallas_call(
        _tc_b_body,
        grid=(_GRIDP,),
        in_specs=[
            pl.BlockSpec((_NC, _BLKP, 128), lambda i: (0, i, 0)),
            pl.BlockSpec((_NC, _BLKP, 128), lambda i: (0, i, 0)),
            pl.BlockSpec((_BLKP, 128), lambda i: (i, 0)),
            pl.BlockSpec((2, 128), lambda i: (0, 0)),
            pl.BlockSpec((4, 128, 128), lambda i: (0, 0, 0)),
        ],
        out_specs=pl.BlockSpec((_NC, _BLKP, 128), lambda i: (0, i, 0)),
        out_shape=jax.ShapeDtypeStruct((_NC, _NPB, 128), F32),
    )(acc, hwp, dinvp, bt, wk)


def _tc_seg_body(acc_ref, hwp_ref, dinv_ref, bat_ref, b_ref, d0w_ref,
                 d0b_ref, dw_ref, db_ref, fw_ref, fb_ref, out_ref, smax_ref):
    i = pl.program_id(0)

    @pl.when(i == 0)
    def _():
        smax_ref[:] = jnp.full((_G + 8, _C), NEG_INF, F32)

    _, h0, h1 = _layer_h(acc_ref, hwp_ref, dinv_ref, b_ref)

    bi = bat_ref[:]
    g_first = bat_ref[0, 0]
    g_last = bat_ref[_BLKP - 1, 127]

    def upd(g, _):
        m0 = jnp.max(jnp.where(bi == g, h0, NEG_INF), axis=0, keepdims=True)
        m1 = jnp.max(jnp.where(bi == g, h1, NEG_INF), axis=0, keepdims=True)
        r0 = m0[:, 0:_LANE]
        r1 = m1[:, 0:_LANE]
        for k in range(1, 8):
            r0 = jnp.maximum(r0, m0[:, k * _LANE:(k + 1) * _LANE])
            r1 = jnp.maximum(r1, m1[:, k * _LANE:(k + 1) * _LANE])
        m = jnp.concatenate([r0, r1], axis=1)
        cur = smax_ref[pl.ds(g, 1), :]
        smax_ref[pl.ds(g, 1), :] = jnp.maximum(cur, m)
        return 0
    lax.fori_loop(g_first, g_last + 1, upd, 0)

    @pl.when(i == _GRIDP - 1)
    def _():
        g = smax_ref[0:_G, :]
        g = jnp.maximum(
            jnp.dot(g, d0w_ref[:], preferred_element_type=F32) + d0b_ref[:],
            0.0)
        for j in range(_DENSE_LAYERS):
            g = jnp.maximum(
                jnp.dot(g, dw_ref[j], preferred_element_type=F32)
                + db_ref[j], 0.0)
        logits = jnp.dot(g, fw_ref[:], preferred_element_type=F32) + fb_ref[:]
        m = jnp.max(logits, axis=1, keepdims=True)
        z = logits - m
        lse = jnp.log(jnp.sum(jnp.exp(z), axis=1, keepdims=True))
        out_ref[:] = (z - lse)[:, 0:2]


def _tc_seg(acc, hwp, dinvp, batpk, bt, d0w, d0b, dw, db, fw, fb):
    return pl.pallas_call(
        _tc_seg_body,
        grid=(_GRIDP,),
        in_specs=[
            pl.BlockSpec((_NC, _BLKP, 128), lambda i: (0, i, 0)),
            pl.BlockSpec((_NC, _BLKP, 128), lambda i: (0, i, 0)),
            pl.BlockSpec((_BLKP, 128), lambda i: (i, 0)),
            pl.BlockSpec((_BLKP, 128), lambda i: (i, 0)),
            pl.BlockSpec((2, 128), lambda i: (0, 0)),
            pl.BlockSpec((_C, _C), lambda i: (0, 0)),
            pl.BlockSpec((1, _C), lambda i: (0, 0)),
            pl.BlockSpec((_DENSE_LAYERS, _C, _C), lambda i: (0, 0, 0)),
            pl.BlockSpec((_DENSE_LAYERS, 1, _C), lambda i: (0, 0, 0)),
            pl.BlockSpec((_C, 8), lambda i: (0, 0)),
            pl.BlockSpec((1, 8), lambda i: (0, 0)),
        ],
        out_specs=pl.BlockSpec((_G, 2), lambda i: (0, 0)),
        out_shape=jax.ShapeDtypeStruct((_G, 2), F32),
        scratch_shapes=[pltpu.VMEM((_G + 8, _C), F32)],
    )(acc, hwp, dinvp, batpk, bt, d0w, d0b, dw, db, fw, fb)


def _pack_scalar(v, pad_value):
    vp = jnp.pad(v, (0, _NP - _N), constant_values=pad_value)
    return jnp.repeat(vp, _LANE).reshape(_NPB, 128)


def kernel(x, edge_index, batch, embed, conv_W, conv_b, dense0_W, dense0_b,
           dense_W, dense_b, final_W, final_b):
    x32 = x.astype(jnp.int32)
    src = edge_index[0].astype(jnp.int32)
    dst = edge_index[1].astype(jnp.int32)
    bat = batch.astype(jnp.int32)

    xpk = _pack_scalar(x32, 0)
    batpk = _pack_scalar(bat, _G)

    srcr = jnp.pad(src, (0, _EP - _E), constant_values=_N).reshape(_RT, 128)
    dstr = jnp.pad(dst, (0, _EP - _E),
                   constant_values=_N).reshape(_RT, 128)

    emb_pad = jnp.zeros((_C, _C), F32).at[:embed.shape[0]].set(embed)

    eye8 = jnp.eye(8, dtype=F32)
    wks = []
    for l in range(1, _CONV_LAYERS):
        w = conv_W[l]
        wks.append(jnp.stack([
            jnp.kron(eye8, w[:_LANE, :_LANE]),
            jnp.kron(eye8, w[:_LANE, _LANE:]),
            jnp.kron(eye8, w[_LANE:, :_LANE]),
            jnp.kron(eye8, w[_LANE:, _LANE:]),
        ]))
    bts = [jnp.stack([jnp.tile(conv_b[l][:_LANE], 8),
                      jnp.tile(conv_b[l][_LANE:], 8)])
           for l in range(_CONV_LAYERS)]

    d0b = dense0_b.reshape(1, _C)
    db = dense_b.reshape(_DENSE_LAYERS, 1, _C)
    fw = jnp.zeros((_C, 8), F32).at[:, :2].set(final_W)
    fb = jnp.full((1, 8), -1e30, F32).at[0, :2].set(final_b)

    degp = _sc_deg(dstr)
    dinvp, hwp = _tc_a(xpk, degp.reshape(_NC, _NPB, 128), emb_pad, conv_W[0])

    for l in range(_CONV_LAYERS):
        acc = _sc_edge(hwp.reshape(_NC, _NP, _LANE), srcr, dstr)
        accp = acc.reshape(_NC, _NPB, 128)
        if l + 1 < _CONV_LAYERS:
            hwp = _tc_b(accp, hwp, dinvp, bts[l], wks[l])
        else:
            out = _tc_seg(accp, hwp, dinvp, batpk, bts[l], dense0_W,
                          d0b, dense_W, db, fw, fb)
    return out


# speedup vs baseline: 47.5891x; 1.0194x over previous
import functools

import jax
import jax.numpy as jnp
from jax import lax
from jax.experimental import pallas as pl
from jax.experimental.pallas import tpu as pltpu
from jax.experimental.pallas import tpu_sc as plsc

F32 = jnp.float32
NEG_INF = float("-inf")

_N = 100000
_E = 1600000
_C = 32
_G = 64
_CONV_LAYERS = 5
_DENSE_LAYERS = 3

_NC = 2
_NS = 16
_LANE = 16

_BLK = 2048
_NP = 100352
_GRID = _NP // _BLK
_NPT = _NP // _NS
_ZCH = 128
_NZ = _NPT // _ZCH

_KCH = 56
_R = 784
_OUTER = _R // _KCH
_RT = _NS * _R
_EP = _RT * 128

_EH = _E // 2
_KCH2 = 56
_R2 = 392
_OUTER2 = _R2 // _KCH2
_RT2 = _NS * _R2
_EP2 = _RT2 * 128


def _fill_rows(ref, nrows, value):
    def body(i, _):
        ref[i, :] = jnp.full((_LANE,), value, F32)
        return 0
    lax.fori_loop(0, nrows, body, 0)


def _sc_mesh():
    return plsc.VectorSubcoreMesh(core_axis_name="c", subcore_axis_name="s")


def _sc_deg(dstr):
    @functools.partial(
        pl.kernel,
        out_type=jax.ShapeDtypeStruct((_NC, _NP, _LANE), F32),
        mesh=_sc_mesh(),
        compiler_params=pltpu.CompilerParams(use_tc_tiling_on_sc=False),
        scratch_types=[
            pltpu.VMEM((_KCH2, 128), jnp.int32),
            pltpu.VMEM((128, _LANE), F32),
            pltpu.VMEM((_ZCH, _LANE), F32),
            pltpu.VMEM_SHARED((_NP, _LANE), F32),
            pltpu.SemaphoreType.DMA,
        ],
    )
    def k(dst_hbm, out_hbm, didx, ones_v, zbuf, acc_sh, dsem):
        c = lax.axis_index("c")
        s = lax.axis_index("s")
        _fill_rows(zbuf, _ZCH, 0.0)
        _fill_rows(ones_v, 128, 1.0)
        base = s * _NPT

        def zero_chunk(m, _):
            pltpu.sync_copy(zbuf, acc_sh.at[pl.ds(base + m * _ZCH, _ZCH)])
            return 0
        lax.fori_loop(0, _NZ, zero_chunk, 0)
        plsc.subcore_barrier()

        rbase = c * (_RT // 2) + s * _R2
        for o in range(_OUTER2):
            pltpu.sync_copy(dst_hbm.at[pl.ds(rbase + o * _KCH2, _KCH2), :],
                            didx)

            def inner(kk, _):
                pltpu.async_copy(ones_v, acc_sh.at[didx.at[kk]], dsem,
                                 add=True)
                return 0
            lax.fori_loop(0, _KCH2, inner, 0)

            def drain(kk, _):
                pltpu.make_async_copy(ones_v, acc_sh.at[didx.at[0]],
                                      dsem).wait()
                return 0
            lax.fori_loop(0, _KCH2, drain, 0)

        plsc.subcore_barrier()

        def copy_out(m, _):
            off = base + m * _ZCH
            pltpu.sync_copy(acc_sh.at[pl.ds(off, _ZCH)], zbuf)
            pltpu.sync_copy(zbuf, out_hbm.at[c, pl.ds(off, _ZCH), :])
            return 0
        lax.fori_loop(0, _NZ, copy_out, 0)

    return k(dstr)


_NBUF = 8
_LA = 7
_NGRP = _KCH // _NBUF


def _sc_edge(hwp, srcr, dstr):
    @functools.partial(
        pl.kernel,
        out_type=jax.ShapeDtypeStruct((_NC, _NP, _LANE), F32),
        mesh=_sc_mesh(),
        compiler_params=pltpu.CompilerParams(use_tc_tiling_on_sc=False),
        scratch_types=[
            pltpu.VMEM((_KCH, 128), jnp.int32),
            pltpu.VMEM((_KCH, 128), jnp.int32),
            pltpu.VMEM((_NBUF, 128, _LANE), F32),
            pltpu.VMEM_SHARED((_NP, _LANE), F32),
            pltpu.SemaphoreType.DMA((_NBUF,)),
            pltpu.SemaphoreType.DMA((_NBUF,)),
        ],
    )
    def k(hwp_hbm, src_hbm, dst_hbm, out_hbm, sidx, didx, rows, acc_sh,
          gsem, ssem):
        c = lax.axis_index("c")
        s = lax.axis_index("s")
        _fill_rows(rows.at[0], _ZCH, 0.0)
        base = s * _NPT

        def zero_chunk(m, _):
            pltpu.sync_copy(rows.at[0],
                            acc_sh.at[pl.ds(base + m * _ZCH, _ZCH)])
            return 0
        lax.fori_loop(0, _NZ, zero_chunk, 0)
        plsc.subcore_barrier()

        def gather(j, b):
            pltpu.async_copy(hwp_hbm.at[c].at[sidx.at[j]], rows.at[b],
                             gsem.at[b])

        def gather_wait(j, b):
            pltpu.make_async_copy(hwp_hbm.at[c].at[sidx.at[j]], rows.at[b],
                                  gsem.at[b]).wait()

        def scat(j, b):
            pltpu.async_copy(rows.at[b], acc_sh.at[didx.at[j]], ssem.at[b],
                             add=True)

        def scat_wait(j, b):
            pltpu.make_async_copy(rows.at[b], acc_sh.at[didx.at[j]],
                                  ssem.at[b]).wait()

        rbase = s * _R
        for o in range(_OUTER):
            if o > 0:
                for u in range(_NBUF):
                    scat_wait(0, u)
            pltpu.sync_copy(src_hbm.at[pl.ds(rbase + o * _KCH, _KCH), :],
                            sidx)
            pltpu.sync_copy(dst_hbm.at[pl.ds(rbase + o * _KCH, _KCH), :],
                            didx)

            for b in range(_LA):
                gather(b, b)

            for u in range(_NBUF):
                gather_wait(u, u)
                scat(u, u)
                bb = (u + _LA) % _NBUF
                if u >= _NBUF - _LA:
                    scat_wait(0, bb)
                gather(u + _LA, bb)

            def group(g, _):
                for u in range(_NBUF):
                    j = g * _NBUF + u
                    gather_wait(j, u)
                    scat(j, u)
                    bb = (u + _LA) % _NBUF
                    scat_wait(0, bb)
                    gather(j + _LA, bb)
                return 0
            lax.fori_loop(1, _NGRP - 1, group, 0)

            for u in range(_NBUF):
                j = (_NGRP - 1) * _NBUF + u
                gather_wait(j, u)
                scat(j, u)
                if u < _NBUF - _LA:
                    bb = (u + _LA) % _NBUF
                    scat_wait(0, bb)
                    gather(j + _LA, bb)

        for u in range(_NBUF):
            scat_wait(0, u)

        plsc.subcore_barrier()

        def copy_out(m, _):
            off = base + m * _ZCH
            pltpu.sync_copy(acc_sh.at[pl.ds(off, _ZCH)], rows.at[0])
            pltpu.sync_copy(rows.at[0], out_hbm.at[c, pl.ds(off, _ZCH), :])
            return 0
        lax.fori_loop(0, _NZ, copy_out, 0)

    return k(hwp, srcr, dstr)


_NPB = _NP // 8
_BLKP = 448
_GRIDP = _NPB // _BLKP
_FLAV = 17


def _tc_a_body(x_ref, deg_ref, emb_ref, w_ref, dinv_ref, hwp_ref):
    i = pl.program_id(0)
    dp = deg_ref[0] + deg_ref[1]
    row_iota = lax.broadcasted_iota(jnp.int32, (_BLKP, 128), 0)
    lane_iota = lax.broadcasted_iota(jnp.int32, (_BLKP, 128), 1)
    nid = 8 * (i * _BLKP + row_iota) + lane_iota // _LANE
    dinv = jnp.where(nid < _N, lax.rsqrt(dp + 1.0), 0.0)
    ew = jnp.dot(emb_ref[:], w_ref[:], preferred_element_type=F32)
    ew0 = jnp.concatenate([ew[:, :_LANE]] * 8, axis=1)
    ew1 = jnp.concatenate([ew[:, _LANE:]] * 8, axis=1)
    xq = x_ref[:]
    h0 = jnp.zeros((_BLKP, 128), F32)
    h1 = jnp.zeros((_BLKP, 128), F32)
    for f in range(_FLAV):
        sel = xq == f
        h0 = jnp.where(sel, ew0[f:f + 1, :], h0)
        h1 = jnp.where(sel, ew1[f:f + 1, :], h1)
    dinv_ref[:] = dinv
    hwp_ref[0, :, :] = dinv * h0
    hwp_ref[1, :, :] = dinv * h1


def _tc_a(xpk, degp, emb_pad, w0):
    return pl.pallas_call(
        _tc_a_body,
        grid=(_GRIDP,),
        in_specs=[
            pl.BlockSpec((_BLKP, 128), lambda i: (i, 0)),
            pl.BlockSpec((_NC, _BLKP, 128), lambda i: (0, i, 0)),
            pl.BlockSpec((_C, _C), lambda i: (0, 0)),
            pl.BlockSpec((_C, _C), lambda i: (0, 0)),
        ],
        out_specs=[
            pl.BlockSpec((_BLKP, 128), lambda i: (i, 0)),
            pl.BlockSpec((_NC, _BLKP, 128), lambda i: (0, i, 0)),
        ],
        out_shape=[
            jax.ShapeDtypeStruct((_NPB, 128), F32),
            jax.ShapeDtypeStruct((_NC, _NPB, 128), F32),
        ],
    )(xpk, degp, emb_pad, w0)


def _layer_h(acc_ref, hwp_ref, dinv_ref, b_ref):
    dinv = dinv_ref[:]
    h0 = jnp.maximum(dinv * (acc_ref[0] + hwp_ref[0]) + b_ref[0:1, :], 0.0)
    h1 = jnp.maximum(dinv * (acc_ref[1] + hwp_ref[1]) + b_ref[1:2, :], 0.0)
    return dinv, h0, h1


def _tc_b_body(acc_ref, hwp_ref, dinv_ref, b_ref, wk_ref, out_ref):
    dinv, h0, h1 = _layer_h(acc_ref, hwp_ref, dinv_ref, b_ref)
    hw0 = (jnp.dot(h0, wk_ref[0], preferred_element_type=F32)
           + jnp.dot(h1, wk_ref[2], preferred_element_type=F32))
    hw1 = (jnp.dot(h0, wk_ref[1], preferred_element_type=F32)
           + jnp.dot(h1, wk_ref[3], preferred_element_type=F32))
    out_ref[0, :, :] = dinv * hw0
    out_ref[1, :, :] = dinv * hw1


def _tc_b(acc, hwp, dinvp, bt, wk):
    return pl.pallas_call(
        _tc_b_body,
        grid=(_GRIDP,),
        in_specs=[
            pl.BlockSpec((_NC, _BLKP, 128), lambda i: (0, i, 0)),
            pl.BlockSpec((_NC, _BLKP, 128), lambda i: (0, i, 0)),
            pl.BlockSpec((_BLKP, 128), lambda i: (i, 0)),
            pl.BlockSpec((2, 128), lambda i: (0, 0)),
            pl.BlockSpec((4, 128, 128), lambda i: (0, 0, 0)),
        ],
        out_specs=pl.BlockSpec((_NC, _BLKP, 128), lambda i: (0, i, 0)),
        out_shape=jax.ShapeDtypeStruct((_NC, _NPB, 128), F32),
    )(acc, hwp, dinvp, bt, wk)


def _tc_seg_body(acc_ref, hwp_ref, dinv_ref, bat_ref, b_ref, d0w_ref,
                 d0b_ref, dw_ref, db_ref, fw_ref, fb_ref, out_ref, smax_ref):
    i = pl.program_id(0)

    @pl.when(i == 0)
    def _():
        smax_ref[:] = jnp.full((_G + 8, _C), NEG_INF, F32)

    _, h0, h1 = _layer_h(acc_ref, hwp_ref, dinv_ref, b_ref)

    bi = bat_ref[:]
    g_first = bat_ref[0, 0]
    g_last = bat_ref[_BLKP - 1, 127]

    def upd(g, _):
        m0 = jnp.max(jnp.where(bi == g, h0, NEG_INF), axis=0, keepdims=True)
        m1 = jnp.max(jnp.where(bi == g, h1, NEG_INF), axis=0, keepdims=True)
        r0 = m0[:, 0:_LANE]
        r1 = m1[:, 0:_LANE]
        for k in range(1, 8):
            r0 = jnp.maximum(r0, m0[:, k * _LANE:(k + 1) * _LANE])
            r1 = jnp.maximum(r1, m1[:, k * _LANE:(k + 1) * _LANE])
        m = jnp.concatenate([r0, r1], axis=1)
        cur = smax_ref[pl.ds(g, 1), :]
        smax_ref[pl.ds(g, 1), :] = jnp.maximum(cur, m)
        return 0
    lax.fori_loop(g_first, g_last + 1, upd, 0)

    @pl.when(i == _GRIDP - 1)
    def _():
        g = smax_ref[0:_G, :]
        g = jnp.maximum(
            jnp.dot(g, d0w_ref[:], preferred_element_type=F32) + d0b_ref[:],
            0.0)
        for j in range(_DENSE_LAYERS):
            g = jnp.maximum(
                jnp.dot(g, dw_ref[j], preferred_element_type=F32)
                + db_ref[j], 0.0)
        logits = jnp.dot(g, fw_ref[:], preferred_element_type=F32) + fb_ref[:]
        m = jnp.max(logits, axis=1, keepdims=True)
        z = logits - m
        lse = jnp.log(jnp.sum(jnp.exp(z), axis=1, keepdims=True))
        out_ref[:] = (z - lse)[:, 0:2]


def _tc_seg(acc, hwp, dinvp, batpk, bt, d0w, d0b, dw, db, fw, fb):
    return pl.pallas_call(
        _tc_seg_body,
        grid=(_GRIDP,),
        in_specs=[
            pl.BlockSpec((_NC, _BLKP, 128), lambda i: (0, i, 0)),
            pl.BlockSpec((_NC, _BLKP, 128), lambda i: (0, i, 0)),
            pl.BlockSpec((_BLKP, 128), lambda i: (i, 0)),
            pl.BlockSpec((_BLKP, 128), lambda i: (i, 0)),
            pl.BlockSpec((2, 128), lambda i: (0, 0)),
            pl.BlockSpec((_C, _C), lambda i: (0, 0)),
            pl.BlockSpec((1, _C), lambda i: (0, 0)),
            pl.BlockSpec((_DENSE_LAYERS, _C, _C), lambda i: (0, 0, 0)),
            pl.BlockSpec((_DENSE_LAYERS, 1, _C), lambda i: (0, 0, 0)),
            pl.BlockSpec((_C, 8), lambda i: (0, 0)),
            pl.BlockSpec((1, 8), lambda i: (0, 0)),
        ],
        out_specs=pl.BlockSpec((_G, 2), lambda i: (0, 0)),
        out_shape=jax.ShapeDtypeStruct((_G, 2), F32),
        scratch_shapes=[pltpu.VMEM((_G + 8, _C), F32)],
    )(acc, hwp, dinvp, batpk, bt, d0w, d0b, dw, db, fw, fb)


def _pack_scalar(v, pad_value):
    vp = jnp.pad(v, (0, _NP - _N), constant_values=pad_value)
    return jnp.repeat(vp, _LANE).reshape(_NPB, 128)


def kernel(x, edge_index, batch, embed, conv_W, conv_b, dense0_W, dense0_b,
           dense_W, dense_b, final_W, final_b):
    x32 = x.astype(jnp.int32)
    src = edge_index[0].astype(jnp.int32)
    dst = edge_index[1].astype(jnp.int32)
    bat = batch.astype(jnp.int32)

    xpk = _pack_scalar(x32, 0)
    batpk = _pack_scalar(bat, _G)

    srcr = jnp.pad(src, (0, _EP - _E), constant_values=_N).reshape(_RT, 128)
    dstr = jnp.pad(dst, (0, _EP - _E),
                   constant_values=_N).reshape(_RT, 128)

    emb_pad = jnp.zeros((_C, _C), F32).at[:embed.shape[0]].set(embed)

    eye8 = jnp.eye(8, dtype=F32)
    wks = []
    for l in range(1, _CONV_LAYERS):
        w = conv_W[l]
        wks.append(jnp.stack([
            jnp.kron(eye8, w[:_LANE, :_LANE]),
            jnp.kron(eye8, w[:_LANE, _LANE:]),
            jnp.kron(eye8, w[_LANE:, :_LANE]),
            jnp.kron(eye8, w[_LANE:, _LANE:]),
        ]))
    bts = [jnp.stack([jnp.tile(conv_b[l][:_LANE], 8),
                      jnp.tile(conv_b[l][_LANE:], 8)])
           for l in range(_CONV_LAYERS)]

    d0b = dense0_b.reshape(1, _C)
    db = dense_b.reshape(_DENSE_LAYERS, 1, _C)
    fw = jnp.zeros((_C, 8), F32).at[:, :2].set(final_W)
    fb = jnp.full((1, 8), -1e30, F32).at[0, :2].set(final_b)

    degp = _sc_deg(dstr)
    dinvp, hwp = _tc_a(xpk, degp.reshape(_NC, _NPB, 128), emb_pad, conv_W[0])

    for l in range(_CONV_LAYERS):
        acc = _sc_edge(hwp.reshape(_NC, _NP, _LANE), srcr, dstr)
        accp = acc.reshape(_NC, _NPB, 128)
        if l + 1 < _CONV_LAYERS:
            hwp = _tc_b(accp, hwp, dinvp, bts[l], wks[l])
        else:
            out = _tc_seg(accp, hwp, dinvp, batpk, bts[l], dense0_W,
                          d0b, dense_W, db, fw, fb)
    return out


# speedup vs baseline: 47.6305x; 1.0009x over previous
import functools

import jax
import jax.numpy as jnp
from jax import lax
from jax.experimental import pallas as pl
from jax.experimental.pallas import tpu as pltpu
from jax.experimental.pallas import tpu_sc as plsc

F32 = jnp.float32
NEG_INF = float("-inf")

_N = 100000
_E = 1600000
_C = 32
_G = 64
_CONV_LAYERS = 5
_DENSE_LAYERS = 3

_NC = 2
_NS = 16
_LANE = 16

_BLK = 2048
_NP = 100352
_GRID = _NP // _BLK
_NPT = _NP // _NS
_ZCH = 128
_NZ = _NPT // _ZCH

_KCH = 56
_R = 784
_OUTER = _R // _KCH
_RT = _NS * _R
_EP = _RT * 128

_EH = _E // 2
_KCH2 = 56
_R2 = 392
_OUTER2 = _R2 // _KCH2
_RT2 = _NS * _R2
_EP2 = _RT2 * 128


def _fill_rows(ref, nrows, value):
    def body(i, _):
        ref[i, :] = jnp.full((_LANE,), value, F32)
        return 0
    lax.fori_loop(0, nrows, body, 0)


def _sc_mesh():
    return plsc.VectorSubcoreMesh(core_axis_name="c", subcore_axis_name="s")


def _sc_deg(dstr):
    @functools.partial(
        pl.kernel,
        out_type=jax.ShapeDtypeStruct((_NC, _NP, _LANE), F32),
        mesh=_sc_mesh(),
        compiler_params=pltpu.CompilerParams(use_tc_tiling_on_sc=False),
        scratch_types=[
            pltpu.VMEM((_KCH2, 128), jnp.int32),
            pltpu.VMEM((128, _LANE), F32),
            pltpu.VMEM((_ZCH, _LANE), F32),
            pltpu.VMEM_SHARED((_NP, _LANE), F32),
            pltpu.SemaphoreType.DMA,
        ],
    )
    def k(dst_hbm, out_hbm, didx, ones_v, zbuf, acc_sh, dsem):
        c = lax.axis_index("c")
        s = lax.axis_index("s")
        _fill_rows(zbuf, _ZCH, 0.0)
        _fill_rows(ones_v, 128, 1.0)
        base = s * _NPT

        def zero_chunk(m, _):
            pltpu.sync_copy(zbuf, acc_sh.at[pl.ds(base + m * _ZCH, _ZCH)])
            return 0
        lax.fori_loop(0, _NZ, zero_chunk, 0)
        plsc.subcore_barrier()

        rbase = c * (_RT // 2) + s * _R2
        for o in range(_OUTER2):
            pltpu.sync_copy(dst_hbm.at[pl.ds(rbase + o * _KCH2, _KCH2), :],
                            didx)

            def inner(kk, _):
                pltpu.async_copy(ones_v, acc_sh.at[didx.at[kk]], dsem,
                                 add=True)
                return 0
            lax.fori_loop(0, _KCH2, inner, 0)

            def drain(kk, _):
                pltpu.make_async_copy(ones_v, acc_sh.at[didx.at[0]],
                                      dsem).wait()
                return 0
            lax.fori_loop(0, _KCH2, drain, 0)

        plsc.subcore_barrier()

        def copy_out(m, _):
            off = base + m * _ZCH
            pltpu.sync_copy(acc_sh.at[pl.ds(off, _ZCH)], zbuf)
            pltpu.sync_copy(zbuf, out_hbm.at[c, pl.ds(off, _ZCH), :])
            return 0
        lax.fori_loop(0, _NZ, copy_out, 0)

    return k(dstr)


_NBUF = 8
_LA = 7
_NGRP = _KCH // _NBUF


def _sc_edge(hwp, srcr, dstr):
    @functools.partial(
        pl.kernel,
        out_type=jax.ShapeDtypeStruct((_NC, _NP, _LANE), F32),
        mesh=_sc_mesh(),
        compiler_params=pltpu.CompilerParams(use_tc_tiling_on_sc=False),
        scratch_types=[
            pltpu.VMEM((_KCH, 128), jnp.int32),
            pltpu.VMEM((_KCH, 128), jnp.int32),
            pltpu.VMEM((_NBUF, 128, _LANE), F32),
            pltpu.VMEM_SHARED((_NP, _LANE), F32),
            pltpu.SemaphoreType.DMA((_NBUF,)),
            pltpu.SemaphoreType.DMA((_NBUF,)),
        ],
    )
    def k(hwp_hbm, src_hbm, dst_hbm, out_hbm, sidx, didx, rows, acc_sh,
          gsem, ssem):
        c = lax.axis_index("c")
        s = lax.axis_index("s")
        _fill_rows(rows.at[0], _ZCH, 0.0)
        base = s * _NPT

        def zero_chunk(m, _):
            pltpu.sync_copy(rows.at[0],
                            acc_sh.at[pl.ds(base + m * _ZCH, _ZCH)])
            return 0
        lax.fori_loop(0, _NZ, zero_chunk, 0)
        plsc.subcore_barrier()

        def gather(j, b):
            pltpu.async_copy(hwp_hbm.at[c].at[sidx.at[j]], rows.at[b],
                             gsem.at[b])

        def gather_wait(j, b):
            pltpu.make_async_copy(hwp_hbm.at[c].at[sidx.at[j]], rows.at[b],
                                  gsem.at[b]).wait()

        def scat(j, b):
            pltpu.async_copy(rows.at[b], acc_sh.at[didx.at[j]], ssem.at[b],
                             add=True)

        def scat_wait(j, b):
            pltpu.make_async_copy(rows.at[b], acc_sh.at[didx.at[j]],
                                  ssem.at[b]).wait()

        rbase = s * _R
        for o in range(_OUTER):
            if o > 0:
                for u in range(_NBUF):
                    scat_wait(0, u)
            pltpu.sync_copy(src_hbm.at[pl.ds(rbase + o * _KCH, _KCH), :],
                            sidx)
            pltpu.sync_copy(dst_hbm.at[pl.ds(rbase + o * _KCH, _KCH), :],
                            didx)

            for b in range(_LA):
                gather(b, b)

            for u in range(_NBUF):
                gather_wait(u, u)
                scat(u, u)
                bb = (u + _LA) % _NBUF
                if u >= _NBUF - _LA:
                    scat_wait(0, bb)
                gather(u + _LA, bb)

            def group(g, _):
                for u in range(_NBUF):
                    j = g * _NBUF + u
                    gather_wait(j, u)
                    scat(j, u)
                    bb = (u + _LA) % _NBUF
                    scat_wait(0, bb)
                    gather(j + _LA, bb)
                return 0
            lax.fori_loop(1, _NGRP - 1, group, 0)

            for u in range(_NBUF):
                j = (_NGRP - 1) * _NBUF + u
                gather_wait(j, u)
                scat(j, u)
                if u < _NBUF - _LA:
                    bb = (u + _LA) % _NBUF
                    scat_wait(0, bb)
                    gather(j + _LA, bb)

        for u in range(_NBUF):
            scat_wait(0, u)

        plsc.subcore_barrier()

        def copy_out(m, _):
            off = base + m * _ZCH
            pltpu.sync_copy(acc_sh.at[pl.ds(off, _ZCH)], rows.at[0])
            pltpu.sync_copy(rows.at[0], out_hbm.at[c, pl.ds(off, _ZCH), :])
            return 0
        lax.fori_loop(0, _NZ, copy_out, 0)

    return k(hwp, srcr, dstr)


_NPB = _NP // 8
_BLKP = 448
_GRIDP = _NPB // _BLKP
_FLAV = 17


def _tc_a_body(x_ref, deg_ref, emb_ref, w_ref, dinv_ref, hwp_ref):
    i = pl.program_id(0)
    dp = deg_ref[0] + deg_ref[1]
    row_iota = lax.broadcasted_iota(jnp.int32, (_BLKP, 128), 0)
    lane_iota = lax.broadcasted_iota(jnp.int32, (_BLKP, 128), 1)
    nid = 8 * (i * _BLKP + row_iota) + lane_iota // _LANE
    dinv = jnp.where(nid < _N, lax.rsqrt(dp + 1.0), 0.0)
    ew = jnp.dot(emb_ref[:], w_ref[:], preferred_element_type=F32)
    ew0 = jnp.concatenate([ew[:, :_LANE]] * 8, axis=1)
    ew1 = jnp.concatenate([ew[:, _LANE:]] * 8, axis=1)
    xq = x_ref[:]
    h0 = jnp.zeros((_BLKP, 128), F32)
    h1 = jnp.zeros((_BLKP, 128), F32)
    for f in range(_FLAV):
        sel = xq == f
        h0 = jnp.where(sel, ew0[f:f + 1, :], h0)
        h1 = jnp.where(sel, ew1[f:f + 1, :], h1)
    dinv_ref[:] = dinv
    hwp_ref[0, :, :] = dinv * h0
    hwp_ref[1, :, :] = dinv * h1


def _tc_a(xpk, degp, emb_pad, w0):
    return pl.pallas_call(
        _tc_a_body,
        grid=(_GRIDP,),
        in_specs=[
            pl.BlockSpec((_BLKP, 128), lambda i: (i, 0)),
            pl.BlockSpec((_NC, _BLKP, 128), lambda i: (0, i, 0)),
            pl.BlockSpec((_C, _C), lambda i: (0, 0)),
            pl.BlockSpec((_C, _C), lambda i: (0, 0)),
        ],
        out_specs=[
            pl.BlockSpec((_BLKP, 128), lambda i: (i, 0)),
            pl.BlockSpec((_NC, _BLKP, 128), lambda i: (0, i, 0)),
        ],
        out_shape=[
            jax.ShapeDtypeStruct((_NPB, 128), F32),
            jax.ShapeDtypeStruct((_NC, _NPB, 128), F32),
        ],
    )(xpk, degp, emb_pad, w0)


def _layer_h(acc_ref, hwp_ref, dinv_ref, b_ref):
    dinv = dinv_ref[:]
    h0 = jnp.maximum(dinv * (acc_ref[0] + hwp_ref[0]) + b_ref[0:1, :], 0.0)
    h1 = jnp.maximum(dinv * (acc_ref[1] + hwp_ref[1]) + b_ref[1:2, :], 0.0)
    return dinv, h0, h1


def _tc_b_body(acc_ref, hwp_ref, dinv_ref, b_ref, wk_ref, out_ref):
    dinv, h0, h1 = _layer_h(acc_ref, hwp_ref, dinv_ref, b_ref)
    hw0 = (jnp.dot(h0, wk_ref[0], preferred_element_type=F32)
           + jnp.dot(h1, wk_ref[2], preferred_element_type=F32))
    hw1 = (jnp.dot(h0, wk_ref[1], preferred_element_type=F32)
           + jnp.dot(h1, wk_ref[3], preferred_element_type=F32))
    out_ref[0, :, :] = dinv * hw0
    out_ref[1, :, :] = dinv * hw1


def _tc_b(acc, hwp, dinvp, bt, wk):
    return pl.pallas_call(
        _tc_b_body,
        grid=(_GRIDP,),
        in_specs=[
            pl.BlockSpec((_NC, _BLKP, 128), lambda i: (0, i, 0)),
            pl.BlockSpec((_NC, _BLKP, 128), lambda i: (0, i, 0)),
            pl.BlockSpec((_BLKP, 128), lambda i: (i, 0)),
            pl.BlockSpec((2, 128), lambda i: (0, 0)),
            pl.BlockSpec((4, 128, 128), lambda i: (0, 0, 0)),
        ],
        out_specs=pl.BlockSpec((_NC, _BLKP, 128), lambda i: (0, i, 0)),
        out_shape=jax.ShapeDtypeStruct((_NC, _NPB, 128), F32),
    )(acc, hwp, dinvp, bt, wk)


def _tc_seg_body(acc_ref, hwp_ref, dinv_ref, bat_ref, b_ref, d0w_ref,
                 d0b_ref, dw_ref, db_ref, fw_ref, fb_ref, out_ref, smax_ref):
    i = pl.program_id(0)

    @pl.when(i == 0)
    def _():
        smax_ref[:] = jnp.full((_G + 8, _C), NEG_INF, F32)

    _, h0, h1 = _layer_h(acc_ref, hwp_ref, dinv_ref, b_ref)

    bi = bat_ref[:]
    g_first = bat_ref[0, 0]
    g_last = bat_ref[_BLKP - 1, 127]

    def upd(g, _):
        m0 = jnp.max(jnp.where(bi == g, h0, NEG_INF), axis=0, keepdims=True)
        m1 = jnp.max(jnp.where(bi == g, h1, NEG_INF), axis=0, keepdims=True)
        r0 = m0[:, 0:_LANE]
        r1 = m1[:, 0:_LANE]
        for k in range(1, 8):
            r0 = jnp.maximum(r0, m0[:, k * _LANE:(k + 1) * _LANE])
            r1 = jnp.maximum(r1, m1[:, k * _LANE:(k + 1) * _LANE])
        m = jnp.concatenate([r0, r1], axis=1)
        cur = smax_ref[pl.ds(g, 1), :]
        smax_ref[pl.ds(g, 1), :] = jnp.maximum(cur, m)
        return 0
    lax.fori_loop(g_first, g_last + 1, upd, 0)

    @pl.when(i == _GRIDP - 1)
    def _():
        g = smax_ref[0:_G, :]
        g = jnp.maximum(
            jnp.dot(g, d0w_ref[:], preferred_element_type=F32) + d0b_ref[:],
            0.0)
        for j in range(_DENSE_LAYERS):
            g = jnp.maximum(
                jnp.dot(g, dw_ref[j], preferred_element_type=F32)
                + db_ref[j], 0.0)
        logits = jnp.dot(g, fw_ref[:], preferred_element_type=F32) + fb_ref[:]
        m = jnp.max(logits, axis=1, keepdims=True)
        z = logits - m
        lse = jnp.log(jnp.sum(jnp.exp(z), axis=1, keepdims=True))
        out_ref[:] = (z - lse)[:, 0:2]


def _tc_seg(acc, hwp, dinvp, batpk, bt, d0w, d0b, dw, db, fw, fb):
    return pl.pallas_call(
        _tc_seg_body,
        grid=(_GRIDP,),
        in_specs=[
            pl.BlockSpec((_NC, _BLKP, 128), lambda i: (0, i, 0)),
            pl.BlockSpec((_NC, _BLKP, 128), lambda i: (0, i, 0)),
            pl.BlockSpec((_BLKP, 128), lambda i: (i, 0)),
            pl.BlockSpec((_BLKP, 128), lambda i: (i, 0)),
            pl.BlockSpec((2, 128), lambda i: (0, 0)),
            pl.BlockSpec((_C, _C), lambda i: (0, 0)),
            pl.BlockSpec((1, _C), lambda i: (0, 0)),
            pl.BlockSpec((_DENSE_LAYERS, _C, _C), lambda i: (0, 0, 0)),
            pl.BlockSpec((_DENSE_LAYERS, 1, _C), lambda i: (0, 0, 0)),
            pl.BlockSpec((_C, 8), lambda i: (0, 0)),
            pl.BlockSpec((1, 8), lambda i: (0, 0)),
        ],
        out_specs=pl.BlockSpec((_G, 2), lambda i: (0, 0)),
        out_shape=jax.ShapeDtypeStruct((_G, 2), F32),
        scratch_shapes=[pltpu.VMEM((_G + 8, _C), F32)],
    )(acc, hwp, dinvp, batpk, bt, d0w, d0b, dw, db, fw, fb)


def _pack_scalar(v, pad_value):
    vp = jnp.pad(v, (0, _NP - _N), constant_values=pad_value)
    rep = jnp.broadcast_to(vp[:, None], (_NP, _LANE))
    return rep.reshape(_NPB, 128)


def kernel(x, edge_index, batch, embed, conv_W, conv_b, dense0_W, dense0_b,
           dense_W, dense_b, final_W, final_b):
    x32 = x.astype(jnp.int32)
    src = edge_index[0].astype(jnp.int32)
    dst = edge_index[1].astype(jnp.int32)
    bat = batch.astype(jnp.int32)

    xpk = _pack_scalar(x32, 0)
    batpk = _pack_scalar(bat, _G)

    srcr = jnp.pad(src, (0, _EP - _E), constant_values=_N).reshape(_RT, 128)
    dstr = jnp.pad(dst, (0, _EP - _E),
                   constant_values=_N).reshape(_RT, 128)

    emb_pad = jnp.zeros((_C, _C), F32).at[:embed.shape[0]].set(embed)

    eye8 = jnp.eye(8, dtype=F32)
    wks = []
    for l in range(1, _CONV_LAYERS):
        w = conv_W[l]
        wks.append(jnp.stack([
            jnp.kron(eye8, w[:_LANE, :_LANE]),
            jnp.kron(eye8, w[:_LANE, _LANE:]),
            jnp.kron(eye8, w[_LANE:, :_LANE]),
            jnp.kron(eye8, w[_LANE:, _LANE:]),
        ]))
    bts = [jnp.stack([jnp.tile(conv_b[l][:_LANE], 8),
                      jnp.tile(conv_b[l][_LANE:], 8)])
           for l in range(_CONV_LAYERS)]

    d0b = dense0_b.reshape(1, _C)
    db = dense_b.reshape(_DENSE_LAYERS, 1, _C)
    fw = jnp.zeros((_C, 8), F32).at[:, :2].set(final_W)
    fb = jnp.full((1, 8), -1e30, F32).at[0, :2].set(final_b)

    degp = _sc_deg(dstr)
    dinvp, hwp = _tc_a(xpk, degp.reshape(_NC, _NPB, 128), emb_pad, conv_W[0])

    for l in range(_CONV_LAYERS):
        acc = _sc_edge(hwp.reshape(_NC, _NP, _LANE), srcr, dstr)
        accp = acc.reshape(_NC, _NPB, 128)
        if l + 1 < _CONV_LAYERS:
            hwp = _tc_b(accp, hwp, dinvp, bts[l], wks[l])
        else:
            out = _tc_seg(accp, hwp, dinvp, batpk, bts[l], dense0_W,
                          d0b, dense_W, db, fw, fb)
    return out
